# baseline (speedup 1.0000x reference)
import jax
import jax.numpy as jnp
from jax import lax
from jax.experimental import pallas as pl
from jax.experimental.pallas import tpu as pltpu
from jax.experimental.pallas import tpu_sc as plsc

NC = 2
NS = 16
L = 16

V = 100_000
V_PAD = 102_400
F = 200_000
F_PAD = 204_800
NF_TILE = F_PAD // NS
CHUNK = 512
NCHUNK = NF_TILE // CHUNK
SUB = CHUNK // 128
NVB_TILE = V_PAD // NS
NVF_TILE = V // NS
PB = 1600


def _iota16():
    return lax.iota(jnp.int32, L)


def _full16(v):
    return jnp.full((L,), v, dtype=jnp.int32)


def _rsqrt(s):
    i = plsc.bitcast(s, jnp.int32)
    i = 0x5F3759DF - lax.shift_right_arithmetic(i, 1)
    y = plsc.bitcast(i, jnp.float32)
    h = 0.5 * s
    for _ in range(3):
        y = y * (1.5 - h * y * y)
    return y


def _sc_body(verts, faces_f, out, areas, table,
             vbuf, bbuf, fbuf, idxg, idxs0, idxs1, idxs2,
             g0, g1, g2, nrm, ar0, ar1, zbuf, acc, sem, zsem):
    c = lax.axis_index("c")
    s = lax.axis_index("s")
    tile_face0 = s * NF_TILE
    iota = _iota16()
    zero_f = jnp.zeros((L,), jnp.float32)
    zero_i = jnp.zeros((L,), jnp.int32)

    def zb(i, _):
        rows = 2 * i + lax.shift_right_logical(iota, 3)
        cols = lax.bitwise_and(iota, _full16(7))
        plsc.store_scatter(zbuf, [rows, cols], zero_f)
        return _
    lax.fori_loop(0, 32, zb, None)

    def zn(i, _):
        rows = i * L + iota
        plsc.store_scatter(nrm, [rows, _full16(3)], zero_f)
        plsc.store_scatter(nrm, [rows, _full16(7)], zero_f)
        return _
    lax.fori_loop(0, CHUNK // L, zn, None)

    zds = []
    for i in range(NVB_TILE // 64):
        zds.append(pltpu.async_copy(
            zbuf, acc.at[pl.ds(s * NVB_TILE + i * 64, 64)], zsem))

    vb = s * NVB_TILE
    for p in range(NVB_TILE // PB):
        for b in (0, 1):
            base = (2 * c + b) * (3 * V_PAD) + (vb + p * PB) * 3
            pltpu.sync_copy(verts.at[pl.ds(base, 3 * PB)], vbuf)

            def bld(i, _):
                rows = i * L + iota
                r3 = 3 * rows
                x = plsc.load_gather(vbuf, [r3])
                y = plsc.load_gather(vbuf, [r3 + 1])
                z = plsc.load_gather(vbuf, [r3 + 2])
                o = _full16(4 * b)
                plsc.store_scatter(bbuf, [rows, o], x)
                plsc.store_scatter(bbuf, [rows, o + 1], y)
                plsc.store_scatter(bbuf, [rows, o + 2], z)
                return _
            lax.fori_loop(0, PB // L, bld, None)

        pltpu.sync_copy(
            bbuf, table.at[pl.ds(c * V_PAD + vb + p * PB, PB)])
    for d in zds:
        d.wait()
    plsc.subcore_barrier()

    coff = c * V_PAD

    def chunk_body(j, _):
        fb = tile_face0 + j * CHUNK
        with jax.named_scope("stage_extract"):
            pltpu.sync_copy(faces_f.at[pl.ds(fb * 3, 3 * CHUNK)], fbuf)

        def ext(i, _):
            rows = i * L + iota
            r3 = 3 * rows
            v0 = plsc.load_gather(fbuf, [r3])
            v1 = plsc.load_gather(fbuf, [r3 + 1])
            v2 = plsc.load_gather(fbuf, [r3 + 2])
            u = i // (128 // L)
            e = (i % (128 // L)) * L + iota
            plsc.store_scatter(idxs0, [_full16(0) + u, e], v0)
            plsc.store_scatter(idxs1, [_full16(0) + u, e], v1)
            plsc.store_scatter(idxs2, [_full16(0) + u, e], v2)
            plsc.store_scatter(idxg, [_full16(0) + u, e], v0 + coff)
            plsc.store_scatter(idxg, [_full16(SUB) + u, e], v1 + coff)
            plsc.store_scatter(idxg, [_full16(2 * SUB) + u, e], v2 + coff)
            return _
        with jax.named_scope("extract"):
            lax.fori_loop(0, CHUNK // L, ext, None)

        with jax.named_scope("gather"):
            descs = []
            for u in range(SUB):
                descs.append(pltpu.async_copy(
                    table.at[idxg.at[u]], g0.at[pl.ds(u * 128, 128)], sem))
                descs.append(pltpu.async_copy(
                    table.at[idxg.at[SUB + u]], g1.at[pl.ds(u * 128, 128)], sem))
                descs.append(pltpu.async_copy(
                    table.at[idxg.at[2 * SUB + u]], g2.at[pl.ds(u * 128, 128)], sem))
            for d in descs:
                d.wait()

        def step(i, _):
            rows = i * L + iota
            for b in (0, 1):
                o = 4 * b
                ax = plsc.load_gather(g0, [rows, _full16(o)])
                ay = plsc.load_gather(g0, [rows, _full16(o + 1)])
                az = plsc.load_gather(g0, [rows, _full16(o + 2)])
                bx = plsc.load_gather(g1, [rows, _full16(o)])
                by = plsc.load_gather(g1, [rows, _full16(o + 1)])
                bz = plsc.load_gather(g1, [rows, _full16(o + 2)])
                cx = plsc.load_gather(g2, [rows, _full16(o)])
                cy = plsc.load_gather(g2, [rows, _full16(o + 1)])
                cz = plsc.load_gather(g2, [rows, _full16(o + 2)])
                e1x, e1y, e1z = bx - ax, by - ay, bz - az
                e2x, e2y, e2z = cx - bx, cy - by, cz - bz
                nx = e1y * e2z - e1z * e2y
                ny = e1z * e2x - e1x * e2z
                nz = e1x * e2y - e1y * e2x
                plsc.store_scatter(nrm, [rows, _full16(o)], nx)
                plsc.store_scatter(nrm, [rows, _full16(o + 1)], ny)
                plsc.store_scatter(nrm, [rows, _full16(o + 2)], nz)
                sq = nx * nx + ny * ny + nz * nz
                area = 0.5 * sq * _rsqrt(sq)
                ar = ar0 if b == 0 else ar1
                ar[pl.ds(i * L, L)] = area
            return _
        with jax.named_scope("compute"):
            lax.fori_loop(0, CHUNK // L, step, None)

        with jax.named_scope("scatter_add"):
            for u in range(SUB):
                sl = pl.ds(u * 128, 128)
                pltpu.sync_copy(nrm.at[sl], acc.at[idxs0.at[u]], add=True)
                pltpu.sync_copy(nrm.at[sl], acc.at[idxs1.at[u]], add=True)
                pltpu.sync_copy(nrm.at[sl], acc.at[idxs2.at[u]], add=True)

        for b in (0, 1):
            ar = ar0 if b == 0 else ar1

            @pl.when(fb + CHUNK <= F)
            def _():
                pltpu.sync_copy(ar, areas.at[2 * c + b, pl.ds(fb, CHUNK)])

            @pl.when(fb == (F // CHUNK) * CHUNK)
            def _():
                rem = F - (F // CHUNK) * CHUNK
                pltpu.sync_copy(ar.at[pl.ds(0, rem)],
                                areas.at[2 * c + b, pl.ds(fb, rem)])
        return _
    lax.fori_loop(0, NCHUNK, chunk_body, None)

    plsc.subcore_barrier()

    vf = s * NVF_TILE
    for q in range(4):
        nq = PB if q < 3 else NVF_TILE - 3 * PB
        pltpu.sync_copy(acc.at[pl.ds(vf + q * PB, nq)],
                        bbuf.at[pl.ds(0, nq)])

        def fstep(i, _):
            rows = i * L + iota
            for b in (0, 1):
                o = 4 * b
                x = plsc.load_gather(bbuf, [rows, _full16(o)])
                y = plsc.load_gather(bbuf, [rows, _full16(o + 1)])
                z = plsc.load_gather(bbuf, [rows, _full16(o + 2)])
                sq = x * x + y * y + z * z
                r = jnp.where(sq >= 1e-12, _rsqrt(sq), 1e6)
                plsc.store_scatter(bbuf, [rows, _full16(o)], x * r)
                plsc.store_scatter(bbuf, [rows, _full16(o + 1)], y * r)
                plsc.store_scatter(bbuf, [rows, _full16(o + 2)], z * r)
            return _
        lax.fori_loop(0, (nq + L - 1) // L, fstep, None)

        for b in (0, 1):
            pltpu.sync_copy(bbuf.at[pl.ds(0, nq), pl.ds(4 * b, 3)],
                            out.at[2 * c + b, pl.ds(vf + q * PB, nq), :])


@jax.jit
def kernel(vertices, faces):
    faces = jnp.squeeze(faces)
    verts_f = jnp.pad(vertices, ((0, 0), (0, V_PAD - V), (0, 0))).reshape(-1)
    faces_f = jnp.pad(faces, ((0, F_PAD - F), (0, 0))).reshape(-1)

    mesh = plsc.VectorSubcoreMesh(core_axis_name="c", subcore_axis_name="s")
    run = pl.kernel(
        _sc_body,
        out_type=(
            jax.ShapeDtypeStruct((4, V, 3), jnp.float32),
            jax.ShapeDtypeStruct((4, F), jnp.float32),
            jax.ShapeDtypeStruct((NC * V_PAD, 8), jnp.float32),
        ),
        mesh=mesh,
        compiler_params=pltpu.CompilerParams(
            use_tc_tiling_on_sc=False, needs_layout_passes=False),
        scratch_types=(
            pltpu.VMEM((3 * PB,), jnp.float32),
            pltpu.VMEM((PB, 8), jnp.float32),
            pltpu.VMEM((3 * CHUNK,), jnp.int32),
            pltpu.VMEM((3 * SUB, 128), jnp.int32),
            pltpu.VMEM((SUB, 128), jnp.int32),
            pltpu.VMEM((SUB, 128), jnp.int32),
            pltpu.VMEM((SUB, 128), jnp.int32),
            pltpu.VMEM((CHUNK, 8), jnp.float32),
            pltpu.VMEM((CHUNK, 8), jnp.float32),
            pltpu.VMEM((CHUNK, 8), jnp.float32),
            pltpu.VMEM((CHUNK, 8), jnp.float32),
            pltpu.VMEM((CHUNK,), jnp.float32),
            pltpu.VMEM((CHUNK,), jnp.float32),
            pltpu.VMEM((64, 8), jnp.float32),
            pltpu.VMEM_SHARED((V_PAD, 8), jnp.float32),
            pltpu.SemaphoreType.DMA,
            pltpu.SemaphoreType.DMA,
        ),
    )
    vectors, areas_out, _ = run(verts_f, faces_f)
    return (vectors, areas_out)


# speedup vs baseline: 1.0002x; 1.0002x over previous
import jax
import jax.numpy as jnp
from jax import lax
from jax.experimental import pallas as pl
from jax.experimental.pallas import tpu as pltpu
from jax.experimental.pallas import tpu_sc as plsc

NC = 2
NS = 16
L = 16

V = 100_000
V_PAD = 102_400
F = 200_000
F_PAD = 204_800
NF_TILE = F_PAD // NS
CHUNK = 512
NCHUNK = NF_TILE // CHUNK
SUB = CHUNK // 128
NVB_TILE = V_PAD // NS
NVF_TILE = V // NS
PB = 1600


def _iota16():
    return lax.iota(jnp.int32, L)


def _full16(v):
    return jnp.full((L,), v, dtype=jnp.int32)


def _rsqrt(s):
    i = plsc.bitcast(s, jnp.int32)
    i = 0x5F3759DF - lax.shift_right_arithmetic(i, 1)
    y = plsc.bitcast(i, jnp.float32)
    h = 0.5 * s
    for _ in range(3):
        y = y * (1.5 - h * y * y)
    return y


def _sc_body(verts, faces_f, out, areas, table,
             vbuf, bbuf, fbuf, idxg, idxs0, idxs1, idxs2,
             g0, g1, g2, nrm, ar0, ar1, zbuf, acc, sem, zsem):
    c = lax.axis_index("c")
    s = lax.axis_index("s")
    tile_face0 = s * NF_TILE
    iota = _iota16()
    zero_f = jnp.zeros((L,), jnp.float32)
    zero_i = jnp.zeros((L,), jnp.int32)

    with jax.named_scope("p0a_zero"):
        def zb(i, _):
            rows = 2 * i + lax.shift_right_logical(iota, 3)
            cols = lax.bitwise_and(iota, _full16(7))
            plsc.store_scatter(zbuf, [rows, cols], zero_f)
            return _
        lax.fori_loop(0, 32, zb, None)

        def zn(i, _):
            rows = i * L + iota
            plsc.store_scatter(nrm, [rows, _full16(3)], zero_f)
            plsc.store_scatter(nrm, [rows, _full16(7)], zero_f)
            return _
        lax.fori_loop(0, CHUNK // L, zn, None)

    with jax.named_scope("p0b_zeroacc_fire"):
        zds = []
        for i in range(NVB_TILE // 64):
            zds.append(pltpu.async_copy(
                zbuf, acc.at[pl.ds(s * NVB_TILE + i * 64, 64)], zsem))

    with jax.named_scope("p0c_build"):
        vb = s * NVB_TILE
        for p in range(NVB_TILE // PB):
            for b in (0, 1):
                base = (2 * c + b) * (3 * V_PAD) + (vb + p * PB) * 3
                pltpu.sync_copy(verts.at[pl.ds(base, 3 * PB)], vbuf)

                def bld(i, _):
                    rows = i * L + iota
                    r3 = 3 * rows
                    x = plsc.load_gather(vbuf, [r3])
                    y = plsc.load_gather(vbuf, [r3 + 1])
                    z = plsc.load_gather(vbuf, [r3 + 2])
                    o = _full16(4 * b)
                    plsc.store_scatter(bbuf, [rows, o], x)
                    plsc.store_scatter(bbuf, [rows, o + 1], y)
                    plsc.store_scatter(bbuf, [rows, o + 2], z)
                    return _
                lax.fori_loop(0, PB // L, bld, None)

            pltpu.sync_copy(
                bbuf, table.at[pl.ds(c * V_PAD + vb + p * PB, PB)])
    with jax.named_scope("p0d_zerowait_barrier"):
        for d in zds:
            d.wait()
        plsc.subcore_barrier()

    coff = c * V_PAD

    def chunk_body(j, _):
        fb = tile_face0 + j * CHUNK
        with jax.named_scope("stage_extract"):
            pltpu.sync_copy(faces_f.at[pl.ds(fb * 3, 3 * CHUNK)], fbuf)

        def ext(i, _):
            rows = i * L + iota
            r3 = 3 * rows
            v0 = plsc.load_gather(fbuf, [r3])
            v1 = plsc.load_gather(fbuf, [r3 + 1])
            v2 = plsc.load_gather(fbuf, [r3 + 2])
            u = i // (128 // L)
            e = (i % (128 // L)) * L + iota
            plsc.store_scatter(idxs0, [_full16(0) + u, e], v0)
            plsc.store_scatter(idxs1, [_full16(0) + u, e], v1)
            plsc.store_scatter(idxs2, [_full16(0) + u, e], v2)
            plsc.store_scatter(idxg, [_full16(0) + u, e], v0 + coff)
            plsc.store_scatter(idxg, [_full16(SUB) + u, e], v1 + coff)
            plsc.store_scatter(idxg, [_full16(2 * SUB) + u, e], v2 + coff)
            return _
        with jax.named_scope("extract"):
            lax.fori_loop(0, CHUNK // L, ext, None)

        with jax.named_scope("gather"):
            descs = []
            for u in range(SUB):
                descs.append(pltpu.async_copy(
                    table.at[idxg.at[u]], g0.at[pl.ds(u * 128, 128)], sem))
                descs.append(pltpu.async_copy(
                    table.at[idxg.at[SUB + u]], g1.at[pl.ds(u * 128, 128)], sem))
                descs.append(pltpu.async_copy(
                    table.at[idxg.at[2 * SUB + u]], g2.at[pl.ds(u * 128, 128)], sem))
            for d in descs:
                d.wait()

        def step(i, _):
            rows = i * L + iota
            for b in (0, 1):
                o = 4 * b
                ax = plsc.load_gather(g0, [rows, _full16(o)])
                ay = plsc.load_gather(g0, [rows, _full16(o + 1)])
                az = plsc.load_gather(g0, [rows, _full16(o + 2)])
                bx = plsc.load_gather(g1, [rows, _full16(o)])
                by = plsc.load_gather(g1, [rows, _full16(o + 1)])
                bz = plsc.load_gather(g1, [rows, _full16(o + 2)])
                cx = plsc.load_gather(g2, [rows, _full16(o)])
                cy = plsc.load_gather(g2, [rows, _full16(o + 1)])
                cz = plsc.load_gather(g2, [rows, _full16(o + 2)])
                e1x, e1y, e1z = bx - ax, by - ay, bz - az
                e2x, e2y, e2z = cx - bx, cy - by, cz - bz
                nx = e1y * e2z - e1z * e2y
                ny = e1z * e2x - e1x * e2z
                nz = e1x * e2y - e1y * e2x
                plsc.store_scatter(nrm, [rows, _full16(o)], nx)
                plsc.store_scatter(nrm, [rows, _full16(o + 1)], ny)
                plsc.store_scatter(nrm, [rows, _full16(o + 2)], nz)
                sq = nx * nx + ny * ny + nz * nz
                area = 0.5 * sq * _rsqrt(sq)
                ar = ar0 if b == 0 else ar1
                ar[pl.ds(i * L, L)] = area
            return _
        with jax.named_scope("compute"):
            lax.fori_loop(0, CHUNK // L, step, None)

        with jax.named_scope("scatter_add"):
            for u in range(SUB):
                sl = pl.ds(u * 128, 128)
                pltpu.sync_copy(nrm.at[sl], acc.at[idxs0.at[u]], add=True)
                pltpu.sync_copy(nrm.at[sl], acc.at[idxs1.at[u]], add=True)
                pltpu.sync_copy(nrm.at[sl], acc.at[idxs2.at[u]], add=True)

        for b in (0, 1):
            ar = ar0 if b == 0 else ar1

            @pl.when(fb + CHUNK <= F)
            def _():
                pltpu.sync_copy(ar, areas.at[2 * c + b, pl.ds(fb, CHUNK)])

            @pl.when(fb == (F // CHUNK) * CHUNK)
            def _():
                rem = F - (F // CHUNK) * CHUNK
                pltpu.sync_copy(ar.at[pl.ds(0, rem)],
                                areas.at[2 * c + b, pl.ds(fb, rem)])
        return _
    lax.fori_loop(0, NCHUNK, chunk_body, None)

    with jax.named_scope("p1end_barrier"):
        plsc.subcore_barrier()

    with jax.named_scope("p2_finalize"):
        vf = s * NVF_TILE
        for q in range(4):
            nq = PB if q < 3 else NVF_TILE - 3 * PB
            pltpu.sync_copy(acc.at[pl.ds(vf + q * PB, nq)],
                            bbuf.at[pl.ds(0, nq)])

            def fstep(i, _):
                rows = i * L + iota
                for b in (0, 1):
                    o = 4 * b
                    x = plsc.load_gather(bbuf, [rows, _full16(o)])
                    y = plsc.load_gather(bbuf, [rows, _full16(o + 1)])
                    z = plsc.load_gather(bbuf, [rows, _full16(o + 2)])
                    sq = x * x + y * y + z * z
                    r = jnp.where(sq >= 1e-12, _rsqrt(sq), 1e6)
                    plsc.store_scatter(bbuf, [rows, _full16(o)], x * r)
                    plsc.store_scatter(bbuf, [rows, _full16(o + 1)], y * r)
                    plsc.store_scatter(bbuf, [rows, _full16(o + 2)], z * r)
                return _
            lax.fori_loop(0, (nq + L - 1) // L, fstep, None)

            for b in (0, 1):
                pltpu.sync_copy(bbuf.at[pl.ds(0, nq), pl.ds(4 * b, 3)],
                                out.at[2 * c + b, pl.ds(vf + q * PB, nq), :])


@jax.jit
def kernel(vertices, faces):
    faces = jnp.squeeze(faces)
    verts_f = jnp.pad(vertices, ((0, 0), (0, V_PAD - V), (0, 0))).reshape(-1)
    faces_f = jnp.pad(faces, ((0, F_PAD - F), (0, 0))).reshape(-1)

    mesh = plsc.VectorSubcoreMesh(core_axis_name="c", subcore_axis_name="s")
    run = pl.kernel(
        _sc_body,
        out_type=(
            jax.ShapeDtypeStruct((4, V, 3), jnp.float32),
            jax.ShapeDtypeStruct((4, F), jnp.float32),
            jax.ShapeDtypeStruct((NC * V_PAD, 8), jnp.float32),
        ),
        mesh=mesh,
        compiler_params=pltpu.CompilerParams(
            use_tc_tiling_on_sc=False, needs_layout_passes=False),
        scratch_types=(
            pltpu.VMEM((3 * PB,), jnp.float32),
            pltpu.VMEM((PB, 8), jnp.float32),
            pltpu.VMEM((3 * CHUNK,), jnp.int32),
            pltpu.VMEM((3 * SUB, 128), jnp.int32),
            pltpu.VMEM((SUB, 128), jnp.int32),
            pltpu.VMEM((SUB, 128), jnp.int32),
            pltpu.VMEM((SUB, 128), jnp.int32),
            pltpu.VMEM((CHUNK, 8), jnp.float32),
            pltpu.VMEM((CHUNK, 8), jnp.float32),
            pltpu.VMEM((CHUNK, 8), jnp.float32),
            pltpu.VMEM((CHUNK, 8), jnp.float32),
            pltpu.VMEM((CHUNK,), jnp.float32),
            pltpu.VMEM((CHUNK,), jnp.float32),
            pltpu.VMEM((64, 8), jnp.float32),
            pltpu.VMEM_SHARED((V_PAD, 8), jnp.float32),
            pltpu.SemaphoreType.DMA,
            pltpu.SemaphoreType.DMA,
        ),
    )
    vectors, areas_out, _ = run(verts_f, faces_f)
    return (vectors, areas_out)


# speedup vs baseline: 1.7568x; 1.7565x over previous
import jax
import jax.numpy as jnp
from jax import lax
from jax.experimental import pallas as pl
from jax.experimental.pallas import tpu as pltpu
from jax.experimental.pallas import tpu_sc as plsc

NC = 2
NS = 16
L = 16

V = 100_000
V_PAD = 102_400
F = 200_000
F_PAD = 204_800
NF_TILE = F_PAD // NS
CHUNK = 512
NCHUNK = NF_TILE // CHUNK
SUB = CHUNK // 128
NV_TILE = V // NS
PB = 1600
PIECES = (PB, PB, PB, NV_TILE - 3 * PB)
F_LAST = (F // CHUNK) * CHUNK
F_REM = F - F_LAST


def _iota16():
    return lax.iota(jnp.int32, L)


def _full16(v):
    return jnp.full((L,), v, dtype=jnp.int32)


def _rsqrt(s):
    i = plsc.bitcast(s, jnp.int32)
    i = 0x5F3759DF - lax.shift_right_arithmetic(i, 1)
    y = plsc.bitcast(i, jnp.float32)
    h = 0.5 * s
    for _ in range(3):
        y = y * (1.5 - h * y * y)
    return y


def _sc_body(verts, faces, out, areas, table,
             vbuf, bbuf, cbuf, fbuf, idxg, idxs0, idxs1, idxs2,
             g0, g1, g2, nrm, ar0, ar1, zbuf, acc, sem, zsem):
    c = lax.axis_index("c")
    s = lax.axis_index("s")
    tile_face0 = s * NF_TILE
    iota = _iota16()
    zero_f = jnp.zeros((L,), jnp.float32)

    def zb(i, _):
        rows = 2 * i + lax.shift_right_logical(iota, 3)
        cols = lax.bitwise_and(iota, _full16(7))
        plsc.store_scatter(zbuf, [rows, cols], zero_f)
        return _
    lax.fori_loop(0, 32, zb, None)

    def zn(i, _):
        rows = i * L + iota
        plsc.store_scatter(nrm, [rows, _full16(3)], zero_f)
        plsc.store_scatter(nrm, [rows, _full16(7)], zero_f)
        return _
    lax.fori_loop(0, CHUNK // L, zn, None)

    zds = []
    for i in range(V_PAD // NS // 64):
        zds.append(pltpu.async_copy(
            zbuf, acc.at[pl.ds(s * (V_PAD // NS) + i * 64, 64)], zsem))

    vb = s * NV_TILE
    for q in range(4):
        nq = PIECES[q]
        row0 = vb + q * PB
        for b in (0, 1):
            pltpu.sync_copy(verts.at[2 * c + b, pl.ds(row0, nq), :],
                            vbuf.at[pl.ds(0, nq), :])

            def bld(i, _):
                rows = i * L + iota
                x = plsc.load_gather(vbuf, [rows, _full16(0)])
                y = plsc.load_gather(vbuf, [rows, _full16(1)])
                z = plsc.load_gather(vbuf, [rows, _full16(2)])
                o = _full16(4 * b)
                plsc.store_scatter(bbuf, [rows, o], x)
                plsc.store_scatter(bbuf, [rows, o + 1], y)
                plsc.store_scatter(bbuf, [rows, o + 2], z)
                return _
            lax.fori_loop(0, (nq + L - 1) // L, bld, None)

        pltpu.sync_copy(bbuf.at[pl.ds(0, nq), :],
                        table.at[pl.ds(c * V_PAD + row0, nq)])
    for d in zds:
        d.wait()
    plsc.subcore_barrier()

    coff = c * V_PAD

    def chunk_body(j, _):
        fb = tile_face0 + j * CHUNK

        @pl.when(fb < F)
        def _do():
            nv = jnp.minimum(F - fb, CHUNK)
            nsteps = nv // L

            @pl.when(nv == CHUNK)
            def _():
                pltpu.sync_copy(faces.at[pl.ds(fb, CHUNK), :], fbuf)

            @pl.when(nv < CHUNK)
            def _():
                pltpu.sync_copy(faces.at[pl.ds(fb, F_REM), :],
                                fbuf.at[pl.ds(0, F_REM), :])

            def ext(i, _):
                rows = i * L + iota
                v0 = plsc.load_gather(fbuf, [rows, _full16(0)])
                v1 = plsc.load_gather(fbuf, [rows, _full16(1)])
                v2 = plsc.load_gather(fbuf, [rows, _full16(2)])
                u = i // (128 // L)
                e = (i % (128 // L)) * L + iota
                plsc.store_scatter(idxs0, [_full16(0) + u, e], v0)
                plsc.store_scatter(idxs1, [_full16(0) + u, e], v1)
                plsc.store_scatter(idxs2, [_full16(0) + u, e], v2)
                plsc.store_scatter(idxg, [_full16(0) + u, e], v0 + coff)
                plsc.store_scatter(idxg, [_full16(SUB) + u, e], v1 + coff)
                plsc.store_scatter(idxg, [_full16(2 * SUB) + u, e], v2 + coff)
                return _
            lax.fori_loop(0, nsteps, ext, None)

            descs = []
            for u in range(SUB):
                descs.append(pltpu.async_copy(
                    table.at[idxg.at[u]], g0.at[pl.ds(u * 128, 128)], sem))
                descs.append(pltpu.async_copy(
                    table.at[idxg.at[SUB + u]], g1.at[pl.ds(u * 128, 128)], sem))
                descs.append(pltpu.async_copy(
                    table.at[idxg.at[2 * SUB + u]], g2.at[pl.ds(u * 128, 128)], sem))
            for d in descs:
                d.wait()

            def step(i, _):
                rows = i * L + iota
                for b in (0, 1):
                    o = 4 * b
                    ax = plsc.load_gather(g0, [rows, _full16(o)])
                    ay = plsc.load_gather(g0, [rows, _full16(o + 1)])
                    az = plsc.load_gather(g0, [rows, _full16(o + 2)])
                    bx = plsc.load_gather(g1, [rows, _full16(o)])
                    by = plsc.load_gather(g1, [rows, _full16(o + 1)])
                    bz = plsc.load_gather(g1, [rows, _full16(o + 2)])
                    cx = plsc.load_gather(g2, [rows, _full16(o)])
                    cy = plsc.load_gather(g2, [rows, _full16(o + 1)])
                    cz = plsc.load_gather(g2, [rows, _full16(o + 2)])
                    e1x, e1y, e1z = bx - ax, by - ay, bz - az
                    e2x, e2y, e2z = cx - bx, cy - by, cz - bz
                    nx = e1y * e2z - e1z * e2y
                    ny = e1z * e2x - e1x * e2z
                    nz = e1x * e2y - e1y * e2x
                    plsc.store_scatter(nrm, [rows, _full16(o)], nx)
                    plsc.store_scatter(nrm, [rows, _full16(o + 1)], ny)
                    plsc.store_scatter(nrm, [rows, _full16(o + 2)], nz)
                    sq = nx * nx + ny * ny + nz * nz
                    area = 0.5 * sq * _rsqrt(sq)
                    ar = ar0 if b == 0 else ar1
                    ar[pl.ds(i * L, L)] = area
                return _
            lax.fori_loop(0, nsteps, step, None)

            @pl.when(nv < CHUNK)
            def _():
                def zt(i, _):
                    rows = F_REM + 2 * i + lax.shift_right_logical(iota, 3)
                    cols = lax.bitwise_and(iota, _full16(7))
                    plsc.store_scatter(nrm, [rows, cols], zero_f)
                    return _
                lax.fori_loop(0, (CHUNK - F_REM) // 2, zt, None)

            for u in range(SUB):
                sl = pl.ds(u * 128, 128)
                pltpu.sync_copy(nrm.at[sl], acc.at[idxs0.at[u]], add=True)
                pltpu.sync_copy(nrm.at[sl], acc.at[idxs1.at[u]], add=True)
                pltpu.sync_copy(nrm.at[sl], acc.at[idxs2.at[u]], add=True)

            for b in (0, 1):
                ar = ar0 if b == 0 else ar1

                @pl.when(nv == CHUNK)
                def _():
                    pltpu.sync_copy(ar, areas.at[2 * c + b, pl.ds(fb, CHUNK)])

                @pl.when(nv < CHUNK)
                def _():
                    pltpu.sync_copy(ar.at[pl.ds(0, F_REM)],
                                    areas.at[2 * c + b, pl.ds(fb, F_REM)])
        return _
    lax.fori_loop(0, NCHUNK, chunk_body, None)

    plsc.subcore_barrier()

    vf = s * NV_TILE
    for q in range(4):
        nq = PIECES[q]
        pltpu.sync_copy(acc.at[pl.ds(vf + q * PB, nq)],
                        bbuf.at[pl.ds(0, nq)])

        for b in (0, 1):
            def fstep(i, _):
                rows = i * L + iota
                o = 4 * b
                x = plsc.load_gather(bbuf, [rows, _full16(o)])
                y = plsc.load_gather(bbuf, [rows, _full16(o + 1)])
                z = plsc.load_gather(bbuf, [rows, _full16(o + 2)])
                sq = x * x + y * y + z * z
                r = jnp.where(sq >= 1e-12, _rsqrt(sq), 1e6)
                plsc.store_scatter(cbuf, [rows, _full16(0)], x * r)
                plsc.store_scatter(cbuf, [rows, _full16(1)], y * r)
                plsc.store_scatter(cbuf, [rows, _full16(2)], z * r)
                return _
            lax.fori_loop(0, (nq + L - 1) // L, fstep, None)

            pltpu.sync_copy(cbuf.at[pl.ds(0, nq), :],
                            out.at[2 * c + b, pl.ds(vf + q * PB, nq), :])


@jax.jit
def kernel(vertices, faces):
    faces = jnp.squeeze(faces)
    mesh = plsc.VectorSubcoreMesh(core_axis_name="c", subcore_axis_name="s")
    run = pl.kernel(
        _sc_body,
        out_type=(
            jax.ShapeDtypeStruct((4, V, 3), jnp.float32),
            jax.ShapeDtypeStruct((4, F), jnp.float32),
            jax.ShapeDtypeStruct((NC * V_PAD, 8), jnp.float32),
        ),
        mesh=mesh,
        compiler_params=pltpu.CompilerParams(
            use_tc_tiling_on_sc=False, needs_layout_passes=False),
        scratch_types=(
            pltpu.VMEM((PB, 3), jnp.float32),
            pltpu.VMEM((PB, 8), jnp.float32),
            pltpu.VMEM((PB, 3), jnp.float32),
            pltpu.VMEM((CHUNK, 3), jnp.int32),
            pltpu.VMEM((3 * SUB, 128), jnp.int32),
            pltpu.VMEM((SUB, 128), jnp.int32),
            pltpu.VMEM((SUB, 128), jnp.int32),
            pltpu.VMEM((SUB, 128), jnp.int32),
            pltpu.VMEM((CHUNK, 8), jnp.float32),
            pltpu.VMEM((CHUNK, 8), jnp.float32),
            pltpu.VMEM((CHUNK, 8), jnp.float32),
            pltpu.VMEM((CHUNK, 8), jnp.float32),
            pltpu.VMEM((CHUNK,), jnp.float32),
            pltpu.VMEM((CHUNK,), jnp.float32),
            pltpu.VMEM((64, 8), jnp.float32),
            pltpu.VMEM_SHARED((V_PAD, 8), jnp.float32),
            pltpu.SemaphoreType.DMA,
            pltpu.SemaphoreType.DMA,
        ),
    )
    vectors, areas_out, _ = run(vertices, faces)
    return (vectors, areas_out)


# speedup vs baseline: 1.8104x; 1.0305x over previous
import jax
import jax.numpy as jnp
from jax import lax
from jax.experimental import pallas as pl
from jax.experimental.pallas import tpu as pltpu
from jax.experimental.pallas import tpu_sc as plsc

NC = 2
NS = 16
L = 16

V = 100_000
V_PAD = 102_400
F = 200_000
F_PAD = 204_800
NF_TILE = F_PAD // NS
CHUNK = 512
NCHUNK = NF_TILE // CHUNK
SUB = CHUNK // 128
NVB = V_PAD // NS
PB = 1600
PB_LAST = 800
ROW_LAST = 99_200
F_LAST = (F // CHUNK) * CHUNK
F_REM = F - F_LAST


def _iota16():
    return lax.iota(jnp.int32, L)


def _full16(v):
    return jnp.full((L,), v, dtype=jnp.int32)


def _rsqrt(s):
    i = plsc.bitcast(s, jnp.int32)
    i = 0x5F3759DF - lax.shift_right_arithmetic(i, 1)
    y = plsc.bitcast(i, jnp.float32)
    h = 0.5 * s
    for _ in range(3):
        y = y * (1.5 - h * y * y)
    return y


def _sc_body(verts, faces, out, areas, table,
             vbuf, bbuf, cbuf, fbuf, idxg, idxs0, idxs1, idxs2,
             g0, g1, g2, nrm, ar0, ar1, zbuf, acc, sem, zsem):
    c = lax.axis_index("c")
    s = lax.axis_index("s")
    tile_face0 = s * NF_TILE
    iota = _iota16()
    zero_f = jnp.zeros((L,), jnp.float32)

    def zb(i, _):
        rows = 2 * i + lax.shift_right_logical(iota, 3)
        cols = lax.bitwise_and(iota, _full16(7))
        plsc.store_scatter(zbuf, [rows, cols], zero_f)
        return _
    lax.fori_loop(0, 32, zb, None)

    def zn(i, _):
        rows = i * L + iota
        plsc.store_scatter(nrm, [rows, _full16(3)], zero_f)
        plsc.store_scatter(nrm, [rows, _full16(7)], zero_f)
        return _
    lax.fori_loop(0, CHUNK // L, zn, None)

    zds = []
    for i in range(NVB // 64):
        zds.append(pltpu.async_copy(
            zbuf, acc.at[pl.ds(s * NVB + i * 64, 64)], zsem))

    for p in range(NVB // PB):
        row0 = s * NVB + p * PB
        nrows = jnp.clip(V - row0, 0, PB)
        for b in (0, 1):
            base = (2 * c + b) * (3 * V) + row0 * 3

            @pl.when(nrows == PB)
            def _():
                pltpu.sync_copy(verts.at[pl.ds(base, 3 * PB)], vbuf)

            @pl.when(nrows == PB_LAST)
            def _():
                pltpu.sync_copy(verts.at[pl.ds(base, 3 * PB_LAST)],
                                vbuf.at[pl.ds(0, 3 * PB_LAST)])

            def bld(i, _):
                rows = i * L + iota
                r3 = 3 * rows
                x = plsc.load_gather(vbuf, [r3])
                y = plsc.load_gather(vbuf, [r3 + 1])
                z = plsc.load_gather(vbuf, [r3 + 2])
                o = _full16(4 * b)
                plsc.store_scatter(bbuf, [rows, o], x)
                plsc.store_scatter(bbuf, [rows, o + 1], y)
                plsc.store_scatter(bbuf, [rows, o + 2], z)
                return _
            lax.fori_loop(0, nrows // L, bld, None)

        @pl.when(nrows == PB)
        def _():
            pltpu.sync_copy(bbuf, table.at[pl.ds(c * V_PAD + row0, PB)])

        @pl.when(nrows == PB_LAST)
        def _():
            pltpu.sync_copy(bbuf.at[pl.ds(0, PB_LAST)],
                            table.at[pl.ds(c * V_PAD + row0, PB_LAST)])
    for d in zds:
        d.wait()
    plsc.subcore_barrier()

    coff = c * V_PAD

    def chunk_body(j, _):
        fb = tile_face0 + j * CHUNK

        @pl.when(fb < F)
        def _do():
            nv = jnp.minimum(F - fb, CHUNK)
            nsteps = nv // L

            @pl.when(nv == CHUNK)
            def _():
                pltpu.sync_copy(faces.at[pl.ds(fb * 3, 3 * CHUNK)], fbuf)

            @pl.when(nv < CHUNK)
            def _():
                pltpu.sync_copy(faces.at[pl.ds(fb * 3, 3 * F_REM)],
                                fbuf.at[pl.ds(0, 3 * F_REM)])

            def ext(i, _):
                rows = i * L + iota
                r3 = 3 * rows
                v0 = plsc.load_gather(fbuf, [r3])
                v1 = plsc.load_gather(fbuf, [r3 + 1])
                v2 = plsc.load_gather(fbuf, [r3 + 2])
                u = i // (128 // L)
                e = (i % (128 // L)) * L + iota
                plsc.store_scatter(idxs0, [_full16(0) + u, e], v0)
                plsc.store_scatter(idxs1, [_full16(0) + u, e], v1)
                plsc.store_scatter(idxs2, [_full16(0) + u, e], v2)
                plsc.store_scatter(idxg, [_full16(0) + u, e], v0 + coff)
                plsc.store_scatter(idxg, [_full16(SUB) + u, e], v1 + coff)
                plsc.store_scatter(idxg, [_full16(2 * SUB) + u, e], v2 + coff)
                return _
            lax.fori_loop(0, nsteps, ext, None)

            descs = []
            for u in range(SUB):
                descs.append(pltpu.async_copy(
                    table.at[idxg.at[u]], g0.at[pl.ds(u * 128, 128)], sem))
                descs.append(pltpu.async_copy(
                    table.at[idxg.at[SUB + u]], g1.at[pl.ds(u * 128, 128)], sem))
                descs.append(pltpu.async_copy(
                    table.at[idxg.at[2 * SUB + u]], g2.at[pl.ds(u * 128, 128)], sem))
            for d in descs:
                d.wait()

            def step(i, _):
                rows = i * L + iota
                for b in (0, 1):
                    o = 4 * b
                    ax = plsc.load_gather(g0, [rows, _full16(o)])
                    ay = plsc.load_gather(g0, [rows, _full16(o + 1)])
                    az = plsc.load_gather(g0, [rows, _full16(o + 2)])
                    bx = plsc.load_gather(g1, [rows, _full16(o)])
                    by = plsc.load_gather(g1, [rows, _full16(o + 1)])
                    bz = plsc.load_gather(g1, [rows, _full16(o + 2)])
                    cx = plsc.load_gather(g2, [rows, _full16(o)])
                    cy = plsc.load_gather(g2, [rows, _full16(o + 1)])
                    cz = plsc.load_gather(g2, [rows, _full16(o + 2)])
                    e1x, e1y, e1z = bx - ax, by - ay, bz - az
                    e2x, e2y, e2z = cx - bx, cy - by, cz - bz
                    nx = e1y * e2z - e1z * e2y
                    ny = e1z * e2x - e1x * e2z
                    nz = e1x * e2y - e1y * e2x
                    plsc.store_scatter(nrm, [rows, _full16(o)], nx)
                    plsc.store_scatter(nrm, [rows, _full16(o + 1)], ny)
                    plsc.store_scatter(nrm, [rows, _full16(o + 2)], nz)
                    sq = nx * nx + ny * ny + nz * nz
                    area = 0.5 * sq * _rsqrt(sq)
                    ar = ar0 if b == 0 else ar1
                    ar[pl.ds(i * L, L)] = area
                return _
            lax.fori_loop(0, nsteps, step, None)

            @pl.when(nv < CHUNK)
            def _():
                def zt(i, _):
                    rows = F_REM + 2 * i + lax.shift_right_logical(iota, 3)
                    cols = lax.bitwise_and(iota, _full16(7))
                    plsc.store_scatter(nrm, [rows, cols], zero_f)
                    return _
                lax.fori_loop(0, (CHUNK - F_REM) // 2, zt, None)

            for u in range(SUB):
                sl = pl.ds(u * 128, 128)
                pltpu.sync_copy(nrm.at[sl], acc.at[idxs0.at[u]], add=True)
                pltpu.sync_copy(nrm.at[sl], acc.at[idxs1.at[u]], add=True)
                pltpu.sync_copy(nrm.at[sl], acc.at[idxs2.at[u]], add=True)

            for b in (0, 1):
                ar = ar0 if b == 0 else ar1
                abase = (2 * c + b) * F + fb

                @pl.when(nv == CHUNK)
                def _():
                    pltpu.sync_copy(ar, areas.at[pl.ds(abase, CHUNK)])

                @pl.when(nv < CHUNK)
                def _():
                    pltpu.sync_copy(ar.at[pl.ds(0, F_REM)],
                                    areas.at[pl.ds(abase, F_REM)])
        return _
    lax.fori_loop(0, NCHUNK, chunk_body, None)

    plsc.subcore_barrier()

    for q in range(NVB // PB):
        row0 = s * NVB + q * PB
        nrows = jnp.clip(V - row0, 0, PB)

        @pl.when(nrows > 0)
        def _fin():
            @pl.when(nrows == PB)
            def _():
                pltpu.sync_copy(acc.at[pl.ds(row0, PB)], bbuf)

            @pl.when(nrows == PB_LAST)
            def _():
                pltpu.sync_copy(acc.at[pl.ds(row0, PB_LAST)],
                                bbuf.at[pl.ds(0, PB_LAST)])

            for b in (0, 1):
                def fstep(i, _):
                    rows = i * L + iota
                    o = 4 * b
                    x = plsc.load_gather(bbuf, [rows, _full16(o)])
                    y = plsc.load_gather(bbuf, [rows, _full16(o + 1)])
                    z = plsc.load_gather(bbuf, [rows, _full16(o + 2)])
                    sq = x * x + y * y + z * z
                    r = jnp.where(sq >= 1e-12, _rsqrt(sq), 1e6)
                    r3 = 3 * rows
                    plsc.store_scatter(cbuf, [r3], x * r)
                    plsc.store_scatter(cbuf, [r3 + 1], y * r)
                    plsc.store_scatter(cbuf, [r3 + 2], z * r)
                    return _
                lax.fori_loop(0, nrows // L, fstep, None)

                obase = (2 * c + b) * (3 * V) + row0 * 3

                @pl.when(nrows == PB)
                def _():
                    pltpu.sync_copy(cbuf, out.at[pl.ds(obase, 3 * PB)])

                @pl.when(nrows == PB_LAST)
                def _():
                    pltpu.sync_copy(cbuf.at[pl.ds(0, 3 * PB_LAST)],
                                    out.at[pl.ds(obase, 3 * PB_LAST)])


@jax.jit
def kernel(vertices, faces):
    faces = jnp.squeeze(faces)
    verts_f = vertices.reshape(-1)
    faces_f = faces.reshape(-1)

    mesh = plsc.VectorSubcoreMesh(core_axis_name="c", subcore_axis_name="s")
    run = pl.kernel(
        _sc_body,
        out_type=(
            jax.ShapeDtypeStruct((4 * V * 3,), jnp.float32),
            jax.ShapeDtypeStruct((4 * F,), jnp.float32),
            jax.ShapeDtypeStruct((NC * V_PAD, 8), jnp.float32),
        ),
        mesh=mesh,
        compiler_params=pltpu.CompilerParams(
            use_tc_tiling_on_sc=False, needs_layout_passes=False),
        scratch_types=(
            pltpu.VMEM((3 * PB,), jnp.float32),
            pltpu.VMEM((PB, 8), jnp.float32),
            pltpu.VMEM((3 * PB,), jnp.float32),
            pltpu.VMEM((3 * CHUNK,), jnp.int32),
            pltpu.VMEM((3 * SUB, 128), jnp.int32),
            pltpu.VMEM((SUB, 128), jnp.int32),
            pltpu.VMEM((SUB, 128), jnp.int32),
            pltpu.VMEM((SUB, 128), jnp.int32),
            pltpu.VMEM((CHUNK, 8), jnp.float32),
            pltpu.VMEM((CHUNK, 8), jnp.float32),
            pltpu.VMEM((CHUNK, 8), jnp.float32),
            pltpu.VMEM((CHUNK, 8), jnp.float32),
            pltpu.VMEM((CHUNK,), jnp.float32),
            pltpu.VMEM((CHUNK,), jnp.float32),
            pltpu.VMEM((64, 8), jnp.float32),
            pltpu.VMEM_SHARED((V_PAD, 8), jnp.float32),
            pltpu.SemaphoreType.DMA,
            pltpu.SemaphoreType.DMA,
        ),
    )
    vec_f, areas_f, _ = run(verts_f, faces_f)
    return (vec_f.reshape(4, V, 3), areas_f.reshape(4, F))


# speedup vs baseline: 2.1370x; 1.1804x over previous
import jax
import jax.numpy as jnp
from jax import lax
from jax.experimental import pallas as pl
from jax.experimental.pallas import tpu as pltpu
from jax.experimental.pallas import tpu_sc as plsc

NC = 2
NS = 16
L = 16

V = 100_000
V_PAD = 102_400
F = 200_000
F_PAD = 204_800
NF_TILE = F_PAD // NS
CHUNK = 512
NCHUNK = NF_TILE // CHUNK
SUB = CHUNK // 128
NVB = V_PAD // NS
PB = 1600
PB_LAST = 800
ROW_LAST = 99_200
F_LAST = (F // CHUNK) * CHUNK
F_REM = F - F_LAST


def _iota16():
    return lax.iota(jnp.int32, L)


def _full16(v):
    return jnp.full((L,), v, dtype=jnp.int32)


def _rsqrt(s):
    i = plsc.bitcast(s, jnp.int32)
    i = 0x5F3759DF - lax.shift_right_arithmetic(i, 1)
    y = plsc.bitcast(i, jnp.float32)
    h = 0.5 * s
    for _ in range(3):
        y = y * (1.5 - h * y * y)
    return y


def _sc_body(verts, faces, out, areas, table,
             vbuf, bbuf, cbuf, fbuf, idxg, idxs0, idxs1, idxs2,
             g0, g1, g2, nrm, ar0, ar1, zbuf, acc, sem, zsem):
    c = lax.axis_index("c")
    s = lax.axis_index("s")
    tile_face0 = s * NF_TILE
    iota = _iota16()
    zero_f = jnp.zeros((L,), jnp.float32)

    def zb(i, _):
        rows = 2 * i + lax.shift_right_logical(iota, 3)
        cols = lax.bitwise_and(iota, _full16(7))
        plsc.store_scatter(zbuf, [rows, cols], zero_f)
        return _
    lax.fori_loop(0, 32, zb, None)

    def zn(i, _):
        rows = i * L + iota
        plsc.store_scatter(nrm, [rows, _full16(3)], zero_f)
        plsc.store_scatter(nrm, [rows, _full16(7)], zero_f)
        return _
    lax.fori_loop(0, CHUNK // L, zn, None)

    zds = []
    for i in range(NVB // 64):
        zds.append(pltpu.async_copy(
            zbuf, acc.at[pl.ds(s * NVB + i * 64, 64)], zsem))

    for p in range(NVB // PB):
        row0 = s * NVB + p * PB
        nrows = jnp.clip(V - row0, 0, PB)
        for b in (0, 1):
            base = (2 * c + b) * (3 * V) + row0 * 3

            @pl.when(nrows == PB)
            def _():
                pltpu.sync_copy(verts.at[pl.ds(base, 3 * PB)], vbuf)

            @pl.when(nrows == PB_LAST)
            def _():
                pltpu.sync_copy(verts.at[pl.ds(base, 3 * PB_LAST)],
                                vbuf.at[pl.ds(0, 3 * PB_LAST)])

            def bld(i, _):
                rows = i * L + iota
                r3 = 3 * rows
                x = plsc.load_gather(vbuf, [r3])
                y = plsc.load_gather(vbuf, [r3 + 1])
                z = plsc.load_gather(vbuf, [r3 + 2])
                o = _full16(4 * b)
                plsc.store_scatter(bbuf, [rows, o], x)
                plsc.store_scatter(bbuf, [rows, o + 1], y)
                plsc.store_scatter(bbuf, [rows, o + 2], z)
                return _
            lax.fori_loop(0, nrows // L, bld, None)

        @pl.when(nrows == PB)
        def _():
            pltpu.sync_copy(bbuf, table.at[pl.ds(c * V_PAD + row0, PB)])

        @pl.when(nrows == PB_LAST)
        def _():
            pltpu.sync_copy(bbuf.at[pl.ds(0, PB_LAST)],
                            table.at[pl.ds(c * V_PAD + row0, PB_LAST)])
    for d in zds:
        d.wait()
    plsc.subcore_barrier()

    coff = c * V_PAD

    def chunk_body(j, _):
        fb = tile_face0 + j * CHUNK

        @pl.when(fb < F)
        def _do():
            nv = jnp.minimum(F - fb, CHUNK)
            nsteps = nv // L

            @pl.when(nv == CHUNK)
            def _():
                pltpu.sync_copy(faces.at[pl.ds(fb * 3, 3 * CHUNK)], fbuf)

            @pl.when(nv < CHUNK)
            def _():
                pltpu.sync_copy(faces.at[pl.ds(fb * 3, 3 * F_REM)],
                                fbuf.at[pl.ds(0, 3 * F_REM)])

            def ext(i, _):
                rows = i * L + iota
                r3 = 3 * rows
                v0 = plsc.load_gather(fbuf, [r3])
                v1 = plsc.load_gather(fbuf, [r3 + 1])
                v2 = plsc.load_gather(fbuf, [r3 + 2])
                u = i // (128 // L)
                e = (i % (128 // L)) * L + iota
                plsc.store_scatter(idxs0, [_full16(0) + u, e], v0)
                plsc.store_scatter(idxs1, [_full16(0) + u, e], v1)
                plsc.store_scatter(idxs2, [_full16(0) + u, e], v2)
                plsc.store_scatter(idxg, [_full16(0) + u, e], v0 + coff)
                plsc.store_scatter(idxg, [_full16(SUB) + u, e], v1 + coff)
                plsc.store_scatter(idxg, [_full16(2 * SUB) + u, e], v2 + coff)
                return _
            lax.fori_loop(0, nsteps, ext, None)

            descs = []
            for u in range(SUB):
                descs.append(pltpu.async_copy(
                    table.at[idxg.at[u]], g0.at[pl.ds(u * 128, 128)], sem))
                descs.append(pltpu.async_copy(
                    table.at[idxg.at[SUB + u]], g1.at[pl.ds(u * 128, 128)], sem))
                descs.append(pltpu.async_copy(
                    table.at[idxg.at[2 * SUB + u]], g2.at[pl.ds(u * 128, 128)], sem))
            for d in descs:
                d.wait()

            def step(i, _):
                rows = i * L + iota
                for b in (0, 1):
                    o = 4 * b
                    ax = plsc.load_gather(g0, [rows, _full16(o)])
                    ay = plsc.load_gather(g0, [rows, _full16(o + 1)])
                    az = plsc.load_gather(g0, [rows, _full16(o + 2)])
                    bx = plsc.load_gather(g1, [rows, _full16(o)])
                    by = plsc.load_gather(g1, [rows, _full16(o + 1)])
                    bz = plsc.load_gather(g1, [rows, _full16(o + 2)])
                    cx = plsc.load_gather(g2, [rows, _full16(o)])
                    cy = plsc.load_gather(g2, [rows, _full16(o + 1)])
                    cz = plsc.load_gather(g2, [rows, _full16(o + 2)])
                    e1x, e1y, e1z = bx - ax, by - ay, bz - az
                    e2x, e2y, e2z = cx - bx, cy - by, cz - bz
                    nx = e1y * e2z - e1z * e2y
                    ny = e1z * e2x - e1x * e2z
                    nz = e1x * e2y - e1y * e2x
                    plsc.store_scatter(nrm, [rows, _full16(o)], nx)
                    plsc.store_scatter(nrm, [rows, _full16(o + 1)], ny)
                    plsc.store_scatter(nrm, [rows, _full16(o + 2)], nz)
                    sq = nx * nx + ny * ny + nz * nz
                    area = 0.5 * sq * _rsqrt(sq)
                    ar = ar0 if b == 0 else ar1
                    ar[pl.ds(i * L, L)] = area
                return _
            lax.fori_loop(0, nsteps, step, None)

            @pl.when(nv < CHUNK)
            def _():
                def zt(i, _):
                    rows = F_REM + 2 * i + lax.shift_right_logical(iota, 3)
                    cols = lax.bitwise_and(iota, _full16(7))
                    plsc.store_scatter(nrm, [rows, cols], zero_f)
                    return _
                lax.fori_loop(0, (CHUNK - F_REM) // 2, zt, None)

            for u in range(SUB):
                sl = pl.ds(u * 128, 128)
                pltpu.sync_copy(nrm.at[sl], acc.at[idxs0.at[u]], add=True)
                pltpu.sync_copy(nrm.at[sl], acc.at[idxs1.at[u]], add=True)
                pltpu.sync_copy(nrm.at[sl], acc.at[idxs2.at[u]], add=True)

            for b in (0, 1):
                ar = ar0 if b == 0 else ar1

                @pl.when(nv == CHUNK)
                def _():
                    pltpu.sync_copy(ar, areas.at[2 * c + b, pl.ds(fb, CHUNK)])

                @pl.when(nv < CHUNK)
                def _():
                    pltpu.sync_copy(ar.at[pl.ds(0, F_REM)],
                                    areas.at[2 * c + b, pl.ds(fb, F_REM)])
        return _
    lax.fori_loop(0, NCHUNK, chunk_body, None)

    plsc.subcore_barrier()

    for q in range(NVB // PB):
        row0 = s * NVB + q * PB
        nrows = jnp.clip(V - row0, 0, PB)

        @pl.when(nrows > 0)
        def _fin():
            @pl.when(nrows == PB)
            def _():
                pltpu.sync_copy(acc.at[pl.ds(row0, PB)], bbuf)

            @pl.when(nrows == PB_LAST)
            def _():
                pltpu.sync_copy(acc.at[pl.ds(row0, PB_LAST)],
                                bbuf.at[pl.ds(0, PB_LAST)])

            for b in (0, 1):
                def fstep(i, _):
                    rows = i * L + iota
                    o = 4 * b
                    x = plsc.load_gather(bbuf, [rows, _full16(o)])
                    y = plsc.load_gather(bbuf, [rows, _full16(o + 1)])
                    z = plsc.load_gather(bbuf, [rows, _full16(o + 2)])
                    sq = x * x + y * y + z * z
                    r = jnp.where(sq >= 1e-12, _rsqrt(sq), 1e6)
                    plsc.store_scatter(cbuf, [rows, _full16(0)], x * r)
                    plsc.store_scatter(cbuf, [rows, _full16(1)], y * r)
                    plsc.store_scatter(cbuf, [rows, _full16(2)], z * r)
                    return _
                lax.fori_loop(0, nrows // L, fstep, None)

                @pl.when(nrows == PB)
                def _():
                    pltpu.sync_copy(
                        cbuf.at[pl.ds(0, PB), :],
                        out.at[2 * c + b, pl.ds(row0, PB), :])

                @pl.when(nrows == PB_LAST)
                def _():
                    pltpu.sync_copy(
                        cbuf.at[pl.ds(0, PB_LAST), :],
                        out.at[2 * c + b, pl.ds(row0, PB_LAST), :])


@jax.jit
def kernel(vertices, faces):
    faces = jnp.squeeze(faces)
    verts_f = vertices.reshape(-1)
    faces_f = faces.reshape(-1)

    mesh = plsc.VectorSubcoreMesh(core_axis_name="c", subcore_axis_name="s")
    run = pl.kernel(
        _sc_body,
        out_type=(
            jax.ShapeDtypeStruct((4, V, 3), jnp.float32),
            jax.ShapeDtypeStruct((4, F), jnp.float32),
            jax.ShapeDtypeStruct((NC * V_PAD, 8), jnp.float32),
        ),
        mesh=mesh,
        compiler_params=pltpu.CompilerParams(
            use_tc_tiling_on_sc=False, needs_layout_passes=False),
        scratch_types=(
            pltpu.VMEM((3 * PB,), jnp.float32),
            pltpu.VMEM((PB, 8), jnp.float32),
            pltpu.VMEM((PB, 3), jnp.float32),
            pltpu.VMEM((3 * CHUNK,), jnp.int32),
            pltpu.VMEM((3 * SUB, 128), jnp.int32),
            pltpu.VMEM((SUB, 128), jnp.int32),
            pltpu.VMEM((SUB, 128), jnp.int32),
            pltpu.VMEM((SUB, 128), jnp.int32),
            pltpu.VMEM((CHUNK, 8), jnp.float32),
            pltpu.VMEM((CHUNK, 8), jnp.float32),
            pltpu.VMEM((CHUNK, 8), jnp.float32),
            pltpu.VMEM((CHUNK, 8), jnp.float32),
            pltpu.VMEM((CHUNK,), jnp.float32),
            pltpu.VMEM((CHUNK,), jnp.float32),
            pltpu.VMEM((64, 8), jnp.float32),
            pltpu.VMEM_SHARED((V_PAD, 8), jnp.float32),
            pltpu.SemaphoreType.DMA,
            pltpu.SemaphoreType.DMA,
        ),
    )
    vectors, areas_out, _ = run(verts_f, faces_f)
    return (vectors, areas_out)


# speedup vs baseline: 2.4079x; 1.1268x over previous
import jax
import jax.numpy as jnp
from jax import lax
from jax.experimental import pallas as pl
from jax.experimental.pallas import tpu as pltpu
from jax.experimental.pallas import tpu_sc as plsc

NC = 2
NS = 16
L = 16

V = 100_000
V_PAD = 102_400
F = 200_000
F_PAD = 204_800
NF_TILE = F_PAD // NS
CHUNK = 512
NCHUNK = NF_TILE // CHUNK
SUB = CHUNK // 128
NVB = V_PAD // NS
PB = 1600
PB_LAST = 800
ROW_LAST = 99_200
F_LAST = (F // CHUNK) * CHUNK
F_REM = F - F_LAST


def _iota16():
    return lax.iota(jnp.int32, L)


def _full16(v):
    return jnp.full((L,), v, dtype=jnp.int32)


def _rsqrt(s):
    i = plsc.bitcast(s, jnp.int32)
    i = 0x5F3759DF - lax.shift_right_arithmetic(i, 1)
    y = plsc.bitcast(i, jnp.float32)
    h = 0.5 * s
    for _ in range(3):
        y = y * (1.5 - h * y * y)
    return y


def _sc_body(verts, faces, out, areas, table,
             vbuf, bbuf, cbuf,
             fbuf0, fbuf1, idxg0, idxg1, idxsA0, idxsA1, idxsB0, idxsB1,
             idxsC0, idxsC1, g0a, g0b, g1a, g1b, g2a, g2b,
             nrm0, nrm1, arA0, arA1, arB0, arB1, zbuf, acc,
             ssem0, ssem1, gsem0, gsem1, csem0, csem1, asem0, asem1, zsem):
    fbufs = (fbuf0, fbuf1)
    idxgs = (idxg0, idxg1)
    idxs0s = (idxsA0, idxsA1)
    idxs1s = (idxsB0, idxsB1)
    idxs2s = (idxsC0, idxsC1)
    g0s, g1s, g2s = (g0a, g0b), (g1a, g1b), (g2a, g2b)
    nrms = (nrm0, nrm1)
    ar0s, ar1s = (arA0, arA1), (arB0, arB1)
    ssems, gsems, csems, asems = ((ssem0, ssem1), (gsem0, gsem1),
                                  (csem0, csem1), (asem0, asem1))
    c = lax.axis_index("c")
    s = lax.axis_index("s")
    tile_face0 = s * NF_TILE
    iota = _iota16()
    zero_f = jnp.zeros((L,), jnp.float32)

    def zb(i, _):
        rows = 2 * i + lax.shift_right_logical(iota, 3)
        cols = lax.bitwise_and(iota, _full16(7))
        plsc.store_scatter(zbuf, [rows, cols], zero_f)
        return _
    lax.fori_loop(0, 32, zb, None)

    def zn(i, _):
        rows = i * L + iota
        for nrm in nrms:
            plsc.store_scatter(nrm, [rows, _full16(3)], zero_f)
            plsc.store_scatter(nrm, [rows, _full16(7)], zero_f)
        return _
    lax.fori_loop(0, CHUNK // L, zn, None)

    zds = []
    for i in range(NVB // 64):
        zds.append(pltpu.async_copy(
            zbuf, acc.at[pl.ds(s * NVB + i * 64, 64)], zsem))

    for p in range(NVB // PB):
        row0 = s * NVB + p * PB
        nrows = jnp.clip(V - row0, 0, PB)
        for b in (0, 1):
            base = (2 * c + b) * (3 * V) + row0 * 3

            @pl.when(nrows == PB)
            def _():
                pltpu.sync_copy(verts.at[pl.ds(base, 3 * PB)], vbuf)

            @pl.when(nrows == PB_LAST)
            def _():
                pltpu.sync_copy(verts.at[pl.ds(base, 3 * PB_LAST)],
                                vbuf.at[pl.ds(0, 3 * PB_LAST)])

            def bld(i, _):
                rows = i * L + iota
                r3 = 3 * rows
                x = plsc.load_gather(vbuf, [r3])
                y = plsc.load_gather(vbuf, [r3 + 1])
                z = plsc.load_gather(vbuf, [r3 + 2])
                o = _full16(4 * b)
                plsc.store_scatter(bbuf, [rows, o], x)
                plsc.store_scatter(bbuf, [rows, o + 1], y)
                plsc.store_scatter(bbuf, [rows, o + 2], z)
                return _
            lax.fori_loop(0, nrows // L, bld, None)

        @pl.when(nrows == PB)
        def _():
            pltpu.sync_copy(bbuf, table.at[pl.ds(c * V_PAD + row0, PB)])

        @pl.when(nrows == PB_LAST)
        def _():
            pltpu.sync_copy(bbuf.at[pl.ds(0, PB_LAST)],
                            table.at[pl.ds(c * V_PAD + row0, PB_LAST)])
    for d in zds:
        d.wait()
    plsc.subcore_barrier()

    coff = c * V_PAD

    def fb_of(j):
        return tile_face0 + j * CHUNK

    def vchunk(j):
        jj = jnp.asarray(j, jnp.int32)
        return jnp.logical_and(jnp.logical_and(jj >= 0, jj < NCHUNK),
                               fb_of(jj) < F)

    def is_full(j):
        return fb_of(j) + CHUNK <= F

    def stage_fire(j, p):
        @pl.when(vchunk(j))
        def _():
            fb3 = fb_of(j) * 3

            @pl.when(is_full(j))
            def _():
                pltpu.async_copy(faces.at[pl.ds(fb3, 3 * CHUNK)],
                                 fbufs[p], ssems[p])

            @pl.when(~is_full(j))
            def _():
                pltpu.async_copy(faces.at[pl.ds(fb3, 3 * F_REM)],
                                 fbufs[p].at[pl.ds(0, 3 * F_REM)], ssems[p])

    def stage_wait(j, p):
        @pl.when(vchunk(j))
        def _():
            @pl.when(is_full(j))
            def _():
                pltpu.make_async_copy(faces.at[pl.ds(0, 3 * CHUNK)],
                                      fbufs[p], ssems[p]).wait()

            @pl.when(~is_full(j))
            def _():
                pltpu.make_async_copy(
                    faces.at[pl.ds(0, 3 * F_REM)],
                    fbufs[p].at[pl.ds(0, 3 * F_REM)], ssems[p]).wait()

    def extract(j, p):
        idxg, idxs0, idxs1, idxs2 = idxgs[p], idxs0s[p], idxs1s[p], idxs2s[p]
        fbuf = fbufs[p]

        @pl.when(vchunk(j))
        def _():
            nsteps = jnp.minimum(F - fb_of(j), CHUNK) // L

            def ext(i, _):
                rows = i * L + iota
                r3 = 3 * rows
                v0 = plsc.load_gather(fbuf, [r3])
                v1 = plsc.load_gather(fbuf, [r3 + 1])
                v2 = plsc.load_gather(fbuf, [r3 + 2])
                u = i // (128 // L)
                e = (i % (128 // L)) * L + iota
                plsc.store_scatter(idxs0, [_full16(0) + u, e], v0)
                plsc.store_scatter(idxs1, [_full16(0) + u, e], v1)
                plsc.store_scatter(idxs2, [_full16(0) + u, e], v2)
                plsc.store_scatter(idxg, [_full16(0) + u, e], v0 + coff)
                plsc.store_scatter(idxg, [_full16(SUB) + u, e], v1 + coff)
                plsc.store_scatter(idxg, [_full16(2 * SUB) + u, e], v2 + coff)
                return _
            lax.fori_loop(0, nsteps, ext, None)

    def gather_fire(j, p):
        idxg = idxgs[p]

        @pl.when(vchunk(j))
        def _():
            for u in range(SUB):
                pltpu.async_copy(table.at[idxg.at[u]],
                                 g0s[p].at[pl.ds(u * 128, 128)], gsems[p])
                pltpu.async_copy(table.at[idxg.at[SUB + u]],
                                 g1s[p].at[pl.ds(u * 128, 128)], gsems[p])
                pltpu.async_copy(table.at[idxg.at[2 * SUB + u]],
                                 g2s[p].at[pl.ds(u * 128, 128)], gsems[p])

    def gather_wait(j, p):
        @pl.when(vchunk(j))
        def _():
            for g in (g0s[p], g1s[p], g2s[p]):
                pltpu.make_async_copy(table.at[pl.ds(0, CHUNK)], g,
                                      gsems[p]).wait()

    def compute(j, p):
        g0, g1, g2 = g0s[p], g1s[p], g2s[p]
        nrm, ar0, ar1 = nrms[p], ar0s[p], ar1s[p]

        @pl.when(vchunk(j))
        def _():
            nsteps = jnp.minimum(F - fb_of(j), CHUNK) // L

            def step(i, _):
                rows = i * L + iota
                for b in (0, 1):
                    o = 4 * b
                    ax = plsc.load_gather(g0, [rows, _full16(o)])
                    ay = plsc.load_gather(g0, [rows, _full16(o + 1)])
                    az = plsc.load_gather(g0, [rows, _full16(o + 2)])
                    bx = plsc.load_gather(g1, [rows, _full16(o)])
                    by = plsc.load_gather(g1, [rows, _full16(o + 1)])
                    bz = plsc.load_gather(g1, [rows, _full16(o + 2)])
                    cx = plsc.load_gather(g2, [rows, _full16(o)])
                    cy = plsc.load_gather(g2, [rows, _full16(o + 1)])
                    cz = plsc.load_gather(g2, [rows, _full16(o + 2)])
                    e1x, e1y, e1z = bx - ax, by - ay, bz - az
                    e2x, e2y, e2z = cx - bx, cy - by, cz - bz
                    nx = e1y * e2z - e1z * e2y
                    ny = e1z * e2x - e1x * e2z
                    nz = e1x * e2y - e1y * e2x
                    plsc.store_scatter(nrm, [rows, _full16(o)], nx)
                    plsc.store_scatter(nrm, [rows, _full16(o + 1)], ny)
                    plsc.store_scatter(nrm, [rows, _full16(o + 2)], nz)
                    sq = nx * nx + ny * ny + nz * nz
                    area = 0.5 * sq * _rsqrt(sq)
                    ar = ar0 if b == 0 else ar1
                    ar[pl.ds(i * L, L)] = area
                return _
            lax.fori_loop(0, nsteps, step, None)

            @pl.when(~is_full(j))
            def _():
                def zt(i, _):
                    rows = F_REM + 2 * i + lax.shift_right_logical(iota, 3)
                    cols = lax.bitwise_and(iota, _full16(7))
                    plsc.store_scatter(nrm, [rows, cols], zero_f)
                    return _
                lax.fori_loop(0, (CHUNK - F_REM) // 2, zt, None)

    def scatter_fire(j, p):
        nrm = nrms[p]
        idxs0, idxs1, idxs2 = idxs0s[p], idxs1s[p], idxs2s[p]

        @pl.when(vchunk(j))
        def _():
            for u in range(SUB):
                sl = pl.ds(u * 128, 128)
                pltpu.async_copy(nrm.at[sl], acc.at[idxs0.at[u]],
                                 csems[p], add=True)
                pltpu.async_copy(nrm.at[sl], acc.at[idxs1.at[u]],
                                 csems[p], add=True)
                pltpu.async_copy(nrm.at[sl], acc.at[idxs2.at[u]],
                                 csems[p], add=True)

    def scatter_wait(j, p):
        @pl.when(vchunk(j))
        def _():
            for _k in range(3):
                pltpu.make_async_copy(table.at[pl.ds(0, CHUNK)], nrms[p],
                                      csems[p]).wait()

    def areas_fire(j, p):
        @pl.when(vchunk(j))
        def _():
            fb = fb_of(j)
            for b in (0, 1):
                ar = ar0s[p] if b == 0 else ar1s[p]

                @pl.when(is_full(j))
                def _():
                    pltpu.async_copy(ar, areas.at[2 * c + b, pl.ds(fb, CHUNK)],
                                     asems[p])

                @pl.when(~is_full(j))
                def _():
                    pltpu.async_copy(ar.at[pl.ds(0, F_REM)],
                                     areas.at[2 * c + b, pl.ds(fb, F_REM)],
                                     asems[p])

    def areas_wait(j, p):
        @pl.when(vchunk(j))
        def _():
            for b in (0, 1):
                ar = ar0s[p] if b == 0 else ar1s[p]

                @pl.when(is_full(j))
                def _():
                    pltpu.make_async_copy(areas.at[0, pl.ds(0, CHUNK)], ar,
                                          asems[p]).wait()

                @pl.when(~is_full(j))
                def _():
                    pltpu.make_async_copy(areas.at[0, pl.ds(0, F_REM)],
                                          ar.at[pl.ds(0, F_REM)],
                                          asems[p]).wait()

    stage_fire(0, 0)
    stage_fire(1, 1)
    stage_wait(0, 0)
    extract(0, 0)
    gather_fire(0, 0)

    def super_body(jj, _):
        for p in (0, 1):
            j = 2 * jj + p
            gather_wait(j, p)
            scatter_wait(j - 1, 1 - p)
            stage_fire(j + 2, p)
            stage_wait(j + 1, 1 - p)
            extract(j + 1, 1 - p)
            gather_fire(j + 1, 1 - p)
            areas_wait(j - 2, p)
            compute(j, p)
            scatter_fire(j, p)
            areas_fire(j, p)
        return _
    lax.fori_loop(0, (NCHUNK + 2) // 2, super_body, None)

    areas_wait(NCHUNK - 1, (NCHUNK - 1) % 2)

    plsc.subcore_barrier()

    for q in range(NVB // PB):
        row0 = s * NVB + q * PB
        nrows = jnp.clip(V - row0, 0, PB)

        @pl.when(nrows > 0)
        def _fin():
            @pl.when(nrows == PB)
            def _():
                pltpu.sync_copy(acc.at[pl.ds(row0, PB)], bbuf)

            @pl.when(nrows == PB_LAST)
            def _():
                pltpu.sync_copy(acc.at[pl.ds(row0, PB_LAST)],
                                bbuf.at[pl.ds(0, PB_LAST)])

            for b in (0, 1):
                def fstep(i, _):
                    rows = i * L + iota
                    o = 4 * b
                    x = plsc.load_gather(bbuf, [rows, _full16(o)])
                    y = plsc.load_gather(bbuf, [rows, _full16(o + 1)])
                    z = plsc.load_gather(bbuf, [rows, _full16(o + 2)])
                    sq = x * x + y * y + z * z
                    r = jnp.where(sq >= 1e-12, _rsqrt(sq), 1e6)
                    plsc.store_scatter(cbuf, [rows, _full16(0)], x * r)
                    plsc.store_scatter(cbuf, [rows, _full16(1)], y * r)
                    plsc.store_scatter(cbuf, [rows, _full16(2)], z * r)
                    return _
                lax.fori_loop(0, nrows // L, fstep, None)

                @pl.when(nrows == PB)
                def _():
                    pltpu.sync_copy(
                        cbuf.at[pl.ds(0, PB), :],
                        out.at[2 * c + b, pl.ds(row0, PB), :])

                @pl.when(nrows == PB_LAST)
                def _():
                    pltpu.sync_copy(
                        cbuf.at[pl.ds(0, PB_LAST), :],
                        out.at[2 * c + b, pl.ds(row0, PB_LAST), :])


@jax.jit
def kernel(vertices, faces):
    faces = jnp.squeeze(faces)
    verts_f = vertices.reshape(-1)
    faces_f = faces.reshape(-1)

    mesh = plsc.VectorSubcoreMesh(core_axis_name="c", subcore_axis_name="s")
    run = pl.kernel(
        _sc_body,
        out_type=(
            jax.ShapeDtypeStruct((4, V, 3), jnp.float32),
            jax.ShapeDtypeStruct((4, F), jnp.float32),
            jax.ShapeDtypeStruct((NC * V_PAD, 8), jnp.float32),
        ),
        mesh=mesh,
        compiler_params=pltpu.CompilerParams(
            use_tc_tiling_on_sc=False, needs_layout_passes=False),
        scratch_types=(
            pltpu.VMEM((3 * PB,), jnp.float32),
            pltpu.VMEM((PB, 8), jnp.float32),
            pltpu.VMEM((PB, 3), jnp.float32),
            pltpu.VMEM((3 * CHUNK,), jnp.int32),
            pltpu.VMEM((3 * CHUNK,), jnp.int32),
            pltpu.VMEM((3 * SUB, 128), jnp.int32),
            pltpu.VMEM((3 * SUB, 128), jnp.int32),
            pltpu.VMEM((SUB, 128), jnp.int32),
            pltpu.VMEM((SUB, 128), jnp.int32),
            pltpu.VMEM((SUB, 128), jnp.int32),
            pltpu.VMEM((SUB, 128), jnp.int32),
            pltpu.VMEM((SUB, 128), jnp.int32),
            pltpu.VMEM((SUB, 128), jnp.int32),
            pltpu.VMEM((CHUNK, 8), jnp.float32),
            pltpu.VMEM((CHUNK, 8), jnp.float32),
            pltpu.VMEM((CHUNK, 8), jnp.float32),
            pltpu.VMEM((CHUNK, 8), jnp.float32),
            pltpu.VMEM((CHUNK, 8), jnp.float32),
            pltpu.VMEM((CHUNK, 8), jnp.float32),
            pltpu.VMEM((CHUNK, 8), jnp.float32),
            pltpu.VMEM((CHUNK, 8), jnp.float32),
            pltpu.VMEM((CHUNK,), jnp.float32),
            pltpu.VMEM((CHUNK,), jnp.float32),
            pltpu.VMEM((CHUNK,), jnp.float32),
            pltpu.VMEM((CHUNK,), jnp.float32),
            pltpu.VMEM((64, 8), jnp.float32),
            pltpu.VMEM_SHARED((V_PAD, 8), jnp.float32),
            pltpu.SemaphoreType.DMA,
            pltpu.SemaphoreType.DMA,
            pltpu.SemaphoreType.DMA,
            pltpu.SemaphoreType.DMA,
            pltpu.SemaphoreType.DMA,
            pltpu.SemaphoreType.DMA,
            pltpu.SemaphoreType.DMA,
            pltpu.SemaphoreType.DMA,
            pltpu.SemaphoreType.DMA,
        ),
    )
    vectors, areas_out, _ = run(verts_f, faces_f)
    return (vectors, areas_out)


# speedup vs baseline: 4.4190x; 1.8352x over previous
import jax
import jax.numpy as jnp
from jax import lax
from jax.experimental import pallas as pl
from jax.experimental.pallas import tpu as pltpu
from jax.experimental.pallas import tpu_sc as plsc

NC = 2
NS = 16
L = 16

V = 100_000
V_PAD = 102_400
F = 200_000
F_PAD = 204_800
NF_TILE = F_PAD // NS
CHUNK = 512
NCHUNK = NF_TILE // CHUNK
SUB = CHUNK // 128
NVB = V_PAD // NS
PB = 1600
PB_LAST = 800
ROW_LAST = 99_200
F_LAST = (F // CHUNK) * CHUNK
F_REM = F - F_LAST


def _iota16():
    return lax.iota(jnp.int32, L)


def _full16(v):
    return jnp.full((L,), v, dtype=jnp.int32)


def _rsqrt(s):
    i = plsc.bitcast(s, jnp.int32)
    i = 0x5F3759DF - lax.shift_right_arithmetic(i, 1)
    y = plsc.bitcast(i, jnp.float32)
    h = 0.5 * s
    for _ in range(3):
        y = y * (1.5 - h * y * y)
    return y


def _sc_body(verts, faces, out, areas, table,
             vbuf, bbuf, cbuf,
             fbuf0, fbuf1, idxg0, idxg1, idxsA0, idxsA1, idxsB0, idxsB1,
             idxsC0, idxsC1, g0a, g0b, g1a, g1b, g2a, g2b,
             nrm0, nrm1, arA0, arA1, arB0, arB1, zbuf, acc,
             ssem0, ssem1, gsem0, gsem1, csem0, csem1, asem0, asem1, zsem):
    fbufs = (fbuf0, fbuf1)
    idxgs = (idxg0, idxg1)
    idxs0s = (idxsA0, idxsA1)
    idxs1s = (idxsB0, idxsB1)
    idxs2s = (idxsC0, idxsC1)
    g0s, g1s, g2s = (g0a, g0b), (g1a, g1b), (g2a, g2b)
    nrms = (nrm0, nrm1)
    ar0s, ar1s = (arA0, arA1), (arB0, arB1)
    ssems, gsems, csems, asems = ((ssem0, ssem1), (gsem0, gsem1),
                                  (csem0, csem1), (asem0, asem1))
    c = lax.axis_index("c")
    s = lax.axis_index("s")
    tile_face0 = s * NF_TILE
    iota = _iota16()
    zero_f = jnp.zeros((L,), jnp.float32)

    def zb(i, _):
        rows = 2 * i + lax.shift_right_logical(iota, 3)
        cols = lax.bitwise_and(iota, _full16(7))
        plsc.store_scatter(zbuf, [rows, cols], zero_f)
        return _
    lax.fori_loop(0, 32, zb, None)

    def zn(i, _):
        rows = i * L + iota
        for nrm in nrms:
            plsc.store_scatter(nrm, [rows, _full16(3)], zero_f)
            plsc.store_scatter(nrm, [rows, _full16(7)], zero_f)
        return _
    lax.fori_loop(0, CHUNK // L, zn, None)

    zds = []
    for i in range(NVB // 64):
        zds.append(pltpu.async_copy(
            zbuf, acc.at[pl.ds(s * NVB + i * 64, 64)], zsem))

    for p in range(NVB // PB):
        row0 = s * NVB + p * PB
        nrows = jnp.clip(V - row0, 0, PB)
        for b in (0, 1):
            for k in range(3):
                base = k * (4 * V) + (2 * c + b) * V + row0

                @pl.when(nrows == PB)
                def _():
                    pltpu.sync_copy(verts.at[pl.ds(base, PB)],
                                    vbuf.at[k])

                @pl.when(nrows == PB_LAST)
                def _():
                    pltpu.sync_copy(verts.at[pl.ds(base, PB_LAST)],
                                    vbuf.at[k, pl.ds(0, PB_LAST)])

            def bld(i, _):
                rows = i * L + iota
                sl = pl.ds(i * L, L)
                x = vbuf[0, sl]
                y = vbuf[1, sl]
                z = vbuf[2, sl]
                o = _full16(4 * b)
                plsc.store_scatter(bbuf, [rows, o], x)
                plsc.store_scatter(bbuf, [rows, o + 1], y)
                plsc.store_scatter(bbuf, [rows, o + 2], z)
                return _
            lax.fori_loop(0, nrows // L, bld, None)

        @pl.when(nrows == PB)
        def _():
            pltpu.sync_copy(bbuf, table.at[pl.ds(c * V_PAD + row0, PB)])

        @pl.when(nrows == PB_LAST)
        def _():
            pltpu.sync_copy(bbuf.at[pl.ds(0, PB_LAST)],
                            table.at[pl.ds(c * V_PAD + row0, PB_LAST)])
    for d in zds:
        d.wait()
    plsc.subcore_barrier()

    coff = c * V_PAD

    def fb_of(j):
        return tile_face0 + j * CHUNK

    def vchunk(j):
        jj = jnp.asarray(j, jnp.int32)
        return jnp.logical_and(jnp.logical_and(jj >= 0, jj < NCHUNK),
                               fb_of(jj) < F)

    def is_full(j):
        return fb_of(j) + CHUNK <= F

    def stage_fire(j, p):
        @pl.when(vchunk(j))
        def _():
            fb = fb_of(j)
            for k in range(3):

                @pl.when(is_full(j))
                def _():
                    pltpu.async_copy(faces.at[pl.ds(k * F + fb, CHUNK)],
                                     fbufs[p].at[k], ssems[p])

                @pl.when(~is_full(j))
                def _():
                    pltpu.async_copy(faces.at[pl.ds(k * F + fb, F_REM)],
                                     fbufs[p].at[k, pl.ds(0, F_REM)], ssems[p])

    def stage_wait(j, p):
        @pl.when(vchunk(j))
        def _():
            for k in range(3):

                @pl.when(is_full(j))
                def _():
                    pltpu.make_async_copy(faces.at[pl.ds(0, CHUNK)],
                                          fbufs[p].at[k], ssems[p]).wait()

                @pl.when(~is_full(j))
                def _():
                    pltpu.make_async_copy(
                        faces.at[pl.ds(0, F_REM)],
                        fbufs[p].at[k, pl.ds(0, F_REM)], ssems[p]).wait()

    def extract(j, p):
        idxg, idxs0, idxs1, idxs2 = idxgs[p], idxs0s[p], idxs1s[p], idxs2s[p]
        fbuf = fbufs[p]

        @pl.when(vchunk(j))
        def _():
            nsteps = jnp.minimum(F - fb_of(j), CHUNK) // L

            def ext(i, _):
                rows = i * L + iota
                sl = pl.ds(i * L, L)
                v0 = fbuf[0, sl]
                v1 = fbuf[1, sl]
                v2 = fbuf[2, sl]
                u = i // (128 // L)
                e = (i % (128 // L)) * L + iota
                plsc.store_scatter(idxs0, [_full16(0) + u, e], v0)
                plsc.store_scatter(idxs1, [_full16(0) + u, e], v1)
                plsc.store_scatter(idxs2, [_full16(0) + u, e], v2)
                plsc.store_scatter(idxg, [_full16(0) + u, e], v0 + coff)
                plsc.store_scatter(idxg, [_full16(SUB) + u, e], v1 + coff)
                plsc.store_scatter(idxg, [_full16(2 * SUB) + u, e], v2 + coff)
                return _
            lax.fori_loop(0, nsteps, ext, None)

    def gather_fire(j, p):
        idxg = idxgs[p]

        @pl.when(vchunk(j))
        def _():
            for u in range(SUB):
                pltpu.async_copy(table.at[idxg.at[u]],
                                 g0s[p].at[pl.ds(u * 128, 128)], gsems[p])
                pltpu.async_copy(table.at[idxg.at[SUB + u]],
                                 g1s[p].at[pl.ds(u * 128, 128)], gsems[p])
                pltpu.async_copy(table.at[idxg.at[2 * SUB + u]],
                                 g2s[p].at[pl.ds(u * 128, 128)], gsems[p])

    def gather_wait(j, p):
        @pl.when(vchunk(j))
        def _():
            for g in (g0s[p], g1s[p], g2s[p]):
                pltpu.make_async_copy(table.at[pl.ds(0, CHUNK)], g,
                                      gsems[p]).wait()

    def compute(j, p):
        g0, g1, g2 = g0s[p], g1s[p], g2s[p]
        nrm, ar0, ar1 = nrms[p], ar0s[p], ar1s[p]

        @pl.when(vchunk(j))
        def _():
            nsteps = jnp.minimum(F - fb_of(j), CHUNK) // L

            def step(i, _):
                rows = i * L + iota
                for b in (0, 1):
                    o = 4 * b
                    ax = plsc.load_gather(g0, [rows, _full16(o)])
                    ay = plsc.load_gather(g0, [rows, _full16(o + 1)])
                    az = plsc.load_gather(g0, [rows, _full16(o + 2)])
                    bx = plsc.load_gather(g1, [rows, _full16(o)])
                    by = plsc.load_gather(g1, [rows, _full16(o + 1)])
                    bz = plsc.load_gather(g1, [rows, _full16(o + 2)])
                    cx = plsc.load_gather(g2, [rows, _full16(o)])
                    cy = plsc.load_gather(g2, [rows, _full16(o + 1)])
                    cz = plsc.load_gather(g2, [rows, _full16(o + 2)])
                    e1x, e1y, e1z = bx - ax, by - ay, bz - az
                    e2x, e2y, e2z = cx - bx, cy - by, cz - bz
                    nx = e1y * e2z - e1z * e2y
                    ny = e1z * e2x - e1x * e2z
                    nz = e1x * e2y - e1y * e2x
                    plsc.store_scatter(nrm, [rows, _full16(o)], nx)
                    plsc.store_scatter(nrm, [rows, _full16(o + 1)], ny)
                    plsc.store_scatter(nrm, [rows, _full16(o + 2)], nz)
                    sq = nx * nx + ny * ny + nz * nz
                    area = 0.5 * sq * _rsqrt(sq)
                    ar = ar0 if b == 0 else ar1
                    ar[pl.ds(i * L, L)] = area
                return _
            lax.fori_loop(0, nsteps, step, None)

            @pl.when(~is_full(j))
            def _():
                def zt(i, _):
                    rows = F_REM + 2 * i + lax.shift_right_logical(iota, 3)
                    cols = lax.bitwise_and(iota, _full16(7))
                    plsc.store_scatter(nrm, [rows, cols], zero_f)
                    return _
                lax.fori_loop(0, (CHUNK - F_REM) // 2, zt, None)

    def scatter_fire(j, p):
        nrm = nrms[p]
        idxs0, idxs1, idxs2 = idxs0s[p], idxs1s[p], idxs2s[p]

        @pl.when(vchunk(j))
        def _():
            for u in range(SUB):
                sl = pl.ds(u * 128, 128)
                pltpu.async_copy(nrm.at[sl], acc.at[idxs0.at[u]],
                                 csems[p], add=True)
                pltpu.async_copy(nrm.at[sl], acc.at[idxs1.at[u]],
                                 csems[p], add=True)
                pltpu.async_copy(nrm.at[sl], acc.at[idxs2.at[u]],
                                 csems[p], add=True)

    def scatter_wait(j, p):
        @pl.when(vchunk(j))
        def _():
            for _k in range(3):
                pltpu.make_async_copy(table.at[pl.ds(0, CHUNK)], nrms[p],
                                      csems[p]).wait()

    def areas_fire(j, p):
        @pl.when(vchunk(j))
        def _():
            fb = fb_of(j)
            for b in (0, 1):
                ar = ar0s[p] if b == 0 else ar1s[p]

                @pl.when(is_full(j))
                def _():
                    pltpu.async_copy(ar, areas.at[2 * c + b, pl.ds(fb, CHUNK)],
                                     asems[p])

                @pl.when(~is_full(j))
                def _():
                    pltpu.async_copy(ar.at[pl.ds(0, F_REM)],
                                     areas.at[2 * c + b, pl.ds(fb, F_REM)],
                                     asems[p])

    def areas_wait(j, p):
        @pl.when(vchunk(j))
        def _():
            for b in (0, 1):
                ar = ar0s[p] if b == 0 else ar1s[p]

                @pl.when(is_full(j))
                def _():
                    pltpu.make_async_copy(areas.at[0, pl.ds(0, CHUNK)], ar,
                                          asems[p]).wait()

                @pl.when(~is_full(j))
                def _():
                    pltpu.make_async_copy(areas.at[0, pl.ds(0, F_REM)],
                                          ar.at[pl.ds(0, F_REM)],
                                          asems[p]).wait()

    stage_fire(0, 0)
    stage_fire(1, 1)
    stage_wait(0, 0)
    extract(0, 0)
    gather_fire(0, 0)

    def super_body(jj, _):
        for p in (0, 1):
            j = 2 * jj + p
            gather_wait(j, p)
            scatter_wait(j - 1, 1 - p)
            stage_fire(j + 2, p)
            stage_wait(j + 1, 1 - p)
            extract(j + 1, 1 - p)
            gather_fire(j + 1, 1 - p)
            areas_wait(j - 2, p)
            compute(j, p)
            scatter_fire(j, p)
            areas_fire(j, p)
        return _
    lax.fori_loop(0, (NCHUNK + 2) // 2, super_body, None)

    areas_wait(NCHUNK - 1, (NCHUNK - 1) % 2)

    plsc.subcore_barrier()

    for q in range(NVB // PB):
        row0 = s * NVB + q * PB
        nrows = jnp.clip(V - row0, 0, PB)

        @pl.when(nrows > 0)
        def _fin():
            @pl.when(nrows == PB)
            def _():
                pltpu.sync_copy(acc.at[pl.ds(row0, PB)], bbuf)

            @pl.when(nrows == PB_LAST)
            def _():
                pltpu.sync_copy(acc.at[pl.ds(row0, PB_LAST)],
                                bbuf.at[pl.ds(0, PB_LAST)])

            for b in (0, 1):
                def fstep(i, _):
                    rows = i * L + iota
                    o = 4 * b
                    x = plsc.load_gather(bbuf, [rows, _full16(o)])
                    y = plsc.load_gather(bbuf, [rows, _full16(o + 1)])
                    z = plsc.load_gather(bbuf, [rows, _full16(o + 2)])
                    sq = x * x + y * y + z * z
                    r = jnp.where(sq >= 1e-12, _rsqrt(sq), 1e6)
                    plsc.store_scatter(cbuf, [rows, _full16(0)], x * r)
                    plsc.store_scatter(cbuf, [rows, _full16(1)], y * r)
                    plsc.store_scatter(cbuf, [rows, _full16(2)], z * r)
                    return _
                lax.fori_loop(0, nrows // L, fstep, None)

                @pl.when(nrows == PB)
                def _():
                    pltpu.sync_copy(
                        cbuf.at[pl.ds(0, PB), :],
                        out.at[2 * c + b, pl.ds(row0, PB), :])

                @pl.when(nrows == PB_LAST)
                def _():
                    pltpu.sync_copy(
                        cbuf.at[pl.ds(0, PB_LAST), :],
                        out.at[2 * c + b, pl.ds(row0, PB_LAST), :])


@jax.jit
def kernel(vertices, faces):
    faces = jnp.squeeze(faces)
    verts_f = vertices.transpose(2, 0, 1).reshape(-1)
    faces_f = faces.T.reshape(-1)

    mesh = plsc.VectorSubcoreMesh(core_axis_name="c", subcore_axis_name="s")
    run = pl.kernel(
        _sc_body,
        out_type=(
            jax.ShapeDtypeStruct((4, V, 3), jnp.float32),
            jax.ShapeDtypeStruct((4, F), jnp.float32),
            jax.ShapeDtypeStruct((NC * V_PAD, 8), jnp.float32),
        ),
        mesh=mesh,
        compiler_params=pltpu.CompilerParams(
            use_tc_tiling_on_sc=False, needs_layout_passes=False),
        scratch_types=(
            pltpu.VMEM((3, PB), jnp.float32),
            pltpu.VMEM((PB, 8), jnp.float32),
            pltpu.VMEM((PB, 3), jnp.float32),
            pltpu.VMEM((3, CHUNK), jnp.int32),
            pltpu.VMEM((3, CHUNK), jnp.int32),
            pltpu.VMEM((3 * SUB, 128), jnp.int32),
            pltpu.VMEM((3 * SUB, 128), jnp.int32),
            pltpu.VMEM((SUB, 128), jnp.int32),
            pltpu.VMEM((SUB, 128), jnp.int32),
            pltpu.VMEM((SUB, 128), jnp.int32),
            pltpu.VMEM((SUB, 128), jnp.int32),
            pltpu.VMEM((SUB, 128), jnp.int32),
            pltpu.VMEM((SUB, 128), jnp.int32),
            pltpu.VMEM((CHUNK, 8), jnp.float32),
            pltpu.VMEM((CHUNK, 8), jnp.float32),
            pltpu.VMEM((CHUNK, 8), jnp.float32),
            pltpu.VMEM((CHUNK, 8), jnp.float32),
            pltpu.VMEM((CHUNK, 8), jnp.float32),
            pltpu.VMEM((CHUNK, 8), jnp.float32),
            pltpu.VMEM((CHUNK, 8), jnp.float32),
            pltpu.VMEM((CHUNK, 8), jnp.float32),
            pltpu.VMEM((CHUNK,), jnp.float32),
            pltpu.VMEM((CHUNK,), jnp.float32),
            pltpu.VMEM((CHUNK,), jnp.float32),
            pltpu.VMEM((CHUNK,), jnp.float32),
            pltpu.VMEM((64, 8), jnp.float32),
            pltpu.VMEM_SHARED((V_PAD, 8), jnp.float32),
            pltpu.SemaphoreType.DMA,
            pltpu.SemaphoreType.DMA,
            pltpu.SemaphoreType.DMA,
            pltpu.SemaphoreType.DMA,
            pltpu.SemaphoreType.DMA,
            pltpu.SemaphoreType.DMA,
            pltpu.SemaphoreType.DMA,
            pltpu.SemaphoreType.DMA,
            pltpu.SemaphoreType.DMA,
        ),
    )
    vectors, areas_out, _ = run(verts_f, faces_f)
    return (vectors, areas_out)


# speedup vs baseline: 4.4232x; 1.0010x over previous
import jax
import jax.numpy as jnp
from jax import lax
from jax.experimental import pallas as pl
from jax.experimental.pallas import tpu as pltpu
from jax.experimental.pallas import tpu_sc as plsc

NC = 2
NS = 16
L = 16

V = 100_000
V_PAD = 102_400
F = 200_000
F_PAD = 204_800
NF_TILE = F_PAD // NS
CHUNK = 512
NCHUNK = NF_TILE // CHUNK
SUB = CHUNK // 128
NVB = V_PAD // NS
PB = 1600
PB_LAST = 800
F_LAST = (F // CHUNK) * CHUNK
F_REM = F - F_LAST


def _iota16():
    return lax.iota(jnp.int32, L)


def _full16(v):
    return jnp.full((L,), v, dtype=jnp.int32)


def _rsqrt(s):
    i = plsc.bitcast(s, jnp.int32)
    i = 0x5F3759DF - lax.shift_right_arithmetic(i, 1)
    y = plsc.bitcast(i, jnp.float32)
    h = 0.5 * s
    for _ in range(3):
        y = y * (1.5 - h * y * y)
    return y


def _sc_body(verts, faces, out, areas, table,
             vbuf, bbuf, cbuf,
             fbuf0, fbuf1, idxg0, idxg1, idxsA0, idxsA1, idxsB0, idxsB1,
             idxsC0, idxsC1, g0a, g0b, g1a, g1b, g2a, g2b,
             nrm0, nrm1, arA0, arA1, arB0, arB1, zbuf, acc,
             ssem0, ssem1, gsem0, gsem1, csem0, csem1, asem0, asem1, zsem):
    fbufs = (fbuf0, fbuf1)
    idxgs = (idxg0, idxg1)
    idxs0s = (idxsA0, idxsA1)
    idxs1s = (idxsB0, idxsB1)
    idxs2s = (idxsC0, idxsC1)
    g0s, g1s, g2s = (g0a, g0b), (g1a, g1b), (g2a, g2b)
    nrms = (nrm0, nrm1)
    ar0s, ar1s = (arA0, arA1), (arB0, arB1)
    ssems, gsems, csems, asems = ((ssem0, ssem1), (gsem0, gsem1),
                                  (csem0, csem1), (asem0, asem1))
    c = lax.axis_index("c")
    s = lax.axis_index("s")
    tile_face0 = s * NF_TILE
    iota = _iota16()
    zero_f = jnp.zeros((L,), jnp.float32)

    def zb(i, _):
        rows = 2 * i + lax.shift_right_logical(iota, 3)
        cols = lax.bitwise_and(iota, _full16(7))
        plsc.store_scatter(zbuf, [rows, cols], zero_f)
        return _
    lax.fori_loop(0, 32, zb, None)

    def zn(i, _):
        rows = i * L + iota
        for nrm in nrms:
            plsc.store_scatter(nrm, [rows, _full16(3)], zero_f)
            plsc.store_scatter(nrm, [rows, _full16(7)], zero_f)
        return _
    lax.fori_loop(0, CHUNK // L, zn, None)

    zds = []
    for i in range(NVB // 64):
        zds.append(pltpu.async_copy(
            zbuf, acc.at[pl.ds(s * NVB + i * 64, 64)], zsem))

    for p in range(NVB // PB):
        row0 = s * NVB + p * PB
        nrows = jnp.clip(V - row0, 0, PB)
        for b in (0, 1):
            for k in range(3):
                base = k * (4 * V) + (2 * c + b) * V + row0

                @pl.when(nrows == PB)
                def _():
                    pltpu.sync_copy(verts.at[pl.ds(base, PB)],
                                    vbuf.at[k])

                @pl.when(nrows == PB_LAST)
                def _():
                    pltpu.sync_copy(verts.at[pl.ds(base, PB_LAST)],
                                    vbuf.at[k, pl.ds(0, PB_LAST)])

            def bld(i, _):
                rows = i * L + iota
                sl = pl.ds(i * L, L)
                x = vbuf[0, sl]
                y = vbuf[1, sl]
                z = vbuf[2, sl]
                o = _full16(4 * b)
                plsc.store_scatter(bbuf, [rows, o], x)
                plsc.store_scatter(bbuf, [rows, o + 1], y)
                plsc.store_scatter(bbuf, [rows, o + 2], z)
                return _
            lax.fori_loop(0, nrows // L, bld, None)

        @pl.when(nrows == PB)
        def _():
            pltpu.sync_copy(bbuf, table.at[pl.ds(c * V_PAD + row0, PB)])

        @pl.when(nrows == PB_LAST)
        def _():
            pltpu.sync_copy(bbuf.at[pl.ds(0, PB_LAST)],
                            table.at[pl.ds(c * V_PAD + row0, PB_LAST)])
    for d in zds:
        d.wait()
    plsc.subcore_barrier()

    coff = c * V_PAD

    def fb_of(j):
        return tile_face0 + j * CHUNK

    def vchunk(j):
        jj = jnp.asarray(j, jnp.int32)
        return jnp.logical_and(jnp.logical_and(jj >= 0, jj < NCHUNK),
                               fb_of(jj) < F)

    def is_full(j):
        return fb_of(j) + CHUNK <= F

    def stage_fire(j, p):
        @pl.when(vchunk(j))
        def _():
            fb = fb_of(j)
            for k in range(3):

                @pl.when(is_full(j))
                def _():
                    pltpu.async_copy(faces.at[pl.ds(k * F + fb, CHUNK)],
                                     fbufs[p].at[k], ssems[p])

                @pl.when(~is_full(j))
                def _():
                    pltpu.async_copy(faces.at[pl.ds(k * F + fb, F_REM)],
                                     fbufs[p].at[k, pl.ds(0, F_REM)], ssems[p])

    def stage_wait(j, p):
        @pl.when(vchunk(j))
        def _():
            for k in range(3):

                @pl.when(is_full(j))
                def _():
                    pltpu.make_async_copy(faces.at[pl.ds(0, CHUNK)],
                                          fbufs[p].at[k], ssems[p]).wait()

                @pl.when(~is_full(j))
                def _():
                    pltpu.make_async_copy(
                        faces.at[pl.ds(0, F_REM)],
                        fbufs[p].at[k, pl.ds(0, F_REM)], ssems[p]).wait()

    def extract(j, p):
        idxg, idxs0, idxs1, idxs2 = idxgs[p], idxs0s[p], idxs1s[p], idxs2s[p]
        fbuf = fbufs[p]

        @pl.when(vchunk(j))
        def _():
            nsteps = jnp.minimum(F - fb_of(j), CHUNK) // L

            def ext(i, _):
                rows = i * L + iota
                sl = pl.ds(i * L, L)
                v0 = fbuf[0, sl]
                v1 = fbuf[1, sl]
                v2 = fbuf[2, sl]
                u = i // (128 // L)
                e = (i % (128 // L)) * L + iota
                plsc.store_scatter(idxs0, [_full16(0) + u, e], v0)
                plsc.store_scatter(idxs1, [_full16(0) + u, e], v1)
                plsc.store_scatter(idxs2, [_full16(0) + u, e], v2)
                plsc.store_scatter(idxg, [_full16(0) + u, e], v0 + coff)
                plsc.store_scatter(idxg, [_full16(SUB) + u, e], v1 + coff)
                plsc.store_scatter(idxg, [_full16(2 * SUB) + u, e], v2 + coff)
                return _
            lax.fori_loop(0, nsteps, ext, None)

    def gather_fire(j, p):
        idxg = idxgs[p]

        @pl.when(vchunk(j))
        def _():
            for u in range(SUB):
                pltpu.async_copy(table.at[idxg.at[u]],
                                 g0s[p].at[pl.ds(u * 128, 128)], gsems[p])
                pltpu.async_copy(table.at[idxg.at[SUB + u]],
                                 g1s[p].at[pl.ds(u * 128, 128)], gsems[p])
                pltpu.async_copy(table.at[idxg.at[2 * SUB + u]],
                                 g2s[p].at[pl.ds(u * 128, 128)], gsems[p])

    def gather_wait(j, p):
        @pl.when(vchunk(j))
        def _():
            for g in (g0s[p], g1s[p], g2s[p]):
                pltpu.make_async_copy(table.at[pl.ds(0, CHUNK)], g,
                                      gsems[p]).wait()

    def compute(j, p):
        g0, g1, g2 = g0s[p], g1s[p], g2s[p]
        nrm, ar0, ar1 = nrms[p], ar0s[p], ar1s[p]

        @pl.when(vchunk(j))
        def _():
            nsteps = jnp.minimum(F - fb_of(j), CHUNK) // L

            def step(i, _):
                rows = i * L + iota
                for b in (0, 1):
                    o = 4 * b
                    ax = plsc.load_gather(g0, [rows, _full16(o)])
                    ay = plsc.load_gather(g0, [rows, _full16(o + 1)])
                    az = plsc.load_gather(g0, [rows, _full16(o + 2)])
                    bx = plsc.load_gather(g1, [rows, _full16(o)])
                    by = plsc.load_gather(g1, [rows, _full16(o + 1)])
                    bz = plsc.load_gather(g1, [rows, _full16(o + 2)])
                    cx = plsc.load_gather(g2, [rows, _full16(o)])
                    cy = plsc.load_gather(g2, [rows, _full16(o + 1)])
                    cz = plsc.load_gather(g2, [rows, _full16(o + 2)])
                    e1x, e1y, e1z = bx - ax, by - ay, bz - az
                    e2x, e2y, e2z = cx - bx, cy - by, cz - bz
                    nx = e1y * e2z - e1z * e2y
                    ny = e1z * e2x - e1x * e2z
                    nz = e1x * e2y - e1y * e2x
                    plsc.store_scatter(nrm, [rows, _full16(o)], nx)
                    plsc.store_scatter(nrm, [rows, _full16(o + 1)], ny)
                    plsc.store_scatter(nrm, [rows, _full16(o + 2)], nz)
                    sq = nx * nx + ny * ny + nz * nz
                    area = 0.5 * sq * _rsqrt(sq)
                    ar = ar0 if b == 0 else ar1
                    ar[pl.ds(i * L, L)] = area
                return _
            lax.fori_loop(0, nsteps, step, None)

            @pl.when(~is_full(j))
            def _():
                def zt(i, _):
                    rows = F_REM + 2 * i + lax.shift_right_logical(iota, 3)
                    cols = lax.bitwise_and(iota, _full16(7))
                    plsc.store_scatter(nrm, [rows, cols], zero_f)
                    return _
                lax.fori_loop(0, (CHUNK - F_REM) // 2, zt, None)

    def scatter_fire(j, p):
        nrm = nrms[p]
        idxs0, idxs1, idxs2 = idxs0s[p], idxs1s[p], idxs2s[p]

        @pl.when(vchunk(j))
        def _():
            for u in range(SUB):
                sl = pl.ds(u * 128, 128)
                pltpu.async_copy(nrm.at[sl], acc.at[idxs0.at[u]],
                                 csems[p], add=True)
                pltpu.async_copy(nrm.at[sl], acc.at[idxs1.at[u]],
                                 csems[p], add=True)
                pltpu.async_copy(nrm.at[sl], acc.at[idxs2.at[u]],
                                 csems[p], add=True)

    def scatter_wait(j, p):
        @pl.when(vchunk(j))
        def _():
            for _k in range(3):
                pltpu.make_async_copy(table.at[pl.ds(0, CHUNK)], nrms[p],
                                      csems[p]).wait()

    def areas_fire(j, p):
        @pl.when(vchunk(j))
        def _():
            fb = fb_of(j)
            for b in (0, 1):
                ar = ar0s[p] if b == 0 else ar1s[p]

                @pl.when(is_full(j))
                def _():
                    pltpu.async_copy(ar, areas.at[2 * c + b, pl.ds(fb, CHUNK)],
                                     asems[p])

                @pl.when(~is_full(j))
                def _():
                    pltpu.async_copy(ar.at[pl.ds(0, F_REM)],
                                     areas.at[2 * c + b, pl.ds(fb, F_REM)],
                                     asems[p])

    def areas_wait(j, p):
        @pl.when(vchunk(j))
        def _():
            for b in (0, 1):
                ar = ar0s[p] if b == 0 else ar1s[p]

                @pl.when(is_full(j))
                def _():
                    pltpu.make_async_copy(areas.at[0, pl.ds(0, CHUNK)], ar,
                                          asems[p]).wait()

                @pl.when(~is_full(j))
                def _():
                    pltpu.make_async_copy(areas.at[0, pl.ds(0, F_REM)],
                                          ar.at[pl.ds(0, F_REM)],
                                          asems[p]).wait()

    stage_fire(0, 0)
    stage_fire(1, 1)
    stage_wait(0, 0)
    extract(0, 0)
    gather_fire(0, 0)

    def super_body(jj, _):
        for p in (0, 1):
            j = 2 * jj + p
            gather_wait(j, p)
            scatter_wait(j - 1, 1 - p)
            stage_fire(j + 2, p)
            stage_wait(j + 1, 1 - p)
            extract(j + 1, 1 - p)
            gather_fire(j + 1, 1 - p)
            areas_wait(j - 2, p)
            compute(j, p)
            scatter_fire(j, p)
            areas_fire(j, p)
        return _
    lax.fori_loop(0, (NCHUNK + 2) // 2, super_body, None)

    areas_wait(NCHUNK - 1, (NCHUNK - 1) % 2)

    plsc.subcore_barrier()

    for q in range(NVB // PB):
        row0 = s * NVB + q * PB
        nrows = jnp.clip(V - row0, 0, PB)

        @pl.when(nrows > 0)
        def _fin():
            @pl.when(nrows == PB)
            def _():
                pltpu.sync_copy(acc.at[pl.ds(row0, PB)], bbuf)

            @pl.when(nrows == PB_LAST)
            def _():
                pltpu.sync_copy(acc.at[pl.ds(row0, PB_LAST)],
                                bbuf.at[pl.ds(0, PB_LAST)])

            for b in (0, 1):
                def fstep(i, _):
                    rows = i * L + iota
                    o = 4 * b
                    x = plsc.load_gather(bbuf, [rows, _full16(o)])
                    y = plsc.load_gather(bbuf, [rows, _full16(o + 1)])
                    z = plsc.load_gather(bbuf, [rows, _full16(o + 2)])
                    sq = x * x + y * y + z * z
                    r = jnp.where(sq >= 1e-12, _rsqrt(sq), 1e6)
                    plsc.store_scatter(cbuf, [rows, _full16(0)], x * r)
                    plsc.store_scatter(cbuf, [rows, _full16(1)], y * r)
                    plsc.store_scatter(cbuf, [rows, _full16(2)], z * r)
                    return _
                lax.fori_loop(0, nrows // L, fstep, None)

                @pl.when(nrows == PB)
                def _():
                    pltpu.sync_copy(
                        cbuf.at[pl.ds(0, PB), :],
                        out.at[2 * c + b, pl.ds(row0, PB), :])

                @pl.when(nrows == PB_LAST)
                def _():
                    pltpu.sync_copy(
                        cbuf.at[pl.ds(0, PB_LAST), :],
                        out.at[2 * c + b, pl.ds(row0, PB_LAST), :])


@jax.jit
def kernel(vertices, faces):
    faces = jnp.squeeze(faces)
    verts_f = vertices.transpose(2, 0, 1).reshape(-1)
    faces_f = faces.T.reshape(-1)

    mesh = plsc.VectorSubcoreMesh(core_axis_name="c", subcore_axis_name="s")
    run = pl.kernel(
        _sc_body,
        out_type=(
            jax.ShapeDtypeStruct((4, V, 3), jnp.float32),
            jax.ShapeDtypeStruct((4, F), jnp.float32),
            jax.ShapeDtypeStruct((NC * V_PAD, 8), jnp.float32),
        ),
        mesh=mesh,
        compiler_params=pltpu.CompilerParams(
            use_tc_tiling_on_sc=False, needs_layout_passes=False),
        scratch_types=(
            pltpu.VMEM((3, PB), jnp.float32),
            pltpu.VMEM((PB, 8), jnp.float32),
            pltpu.VMEM((PB, 3), jnp.float32),
            pltpu.VMEM((3, CHUNK), jnp.int32),
            pltpu.VMEM((3, CHUNK), jnp.int32),
            pltpu.VMEM((3 * SUB, 128), jnp.int32),
            pltpu.VMEM((3 * SUB, 128), jnp.int32),
            pltpu.VMEM((SUB, 128), jnp.int32),
            pltpu.VMEM((SUB, 128), jnp.int32),
            pltpu.VMEM((SUB, 128), jnp.int32),
            pltpu.VMEM((SUB, 128), jnp.int32),
            pltpu.VMEM((SUB, 128), jnp.int32),
            pltpu.VMEM((SUB, 128), jnp.int32),
            pltpu.VMEM((CHUNK, 8), jnp.float32),
            pltpu.VMEM((CHUNK, 8), jnp.float32),
            pltpu.VMEM((CHUNK, 8), jnp.float32),
            pltpu.VMEM((CHUNK, 8), jnp.float32),
            pltpu.VMEM((CHUNK, 8), jnp.float32),
            pltpu.VMEM((CHUNK, 8), jnp.float32),
            pltpu.VMEM((CHUNK, 8), jnp.float32),
            pltpu.VMEM((CHUNK, 8), jnp.float32),
            pltpu.VMEM((CHUNK,), jnp.float32),
            pltpu.VMEM((CHUNK,), jnp.float32),
            pltpu.VMEM((CHUNK,), jnp.float32),
            pltpu.VMEM((CHUNK,), jnp.float32),
            pltpu.VMEM((64, 8), jnp.float32),
            pltpu.VMEM_SHARED((V_PAD, 8), jnp.float32),
            pltpu.SemaphoreType.DMA,
            pltpu.SemaphoreType.DMA,
            pltpu.SemaphoreType.DMA,
            pltpu.SemaphoreType.DMA,
            pltpu.SemaphoreType.DMA,
            pltpu.SemaphoreType.DMA,
            pltpu.SemaphoreType.DMA,
            pltpu.SemaphoreType.DMA,
            pltpu.SemaphoreType.DMA,
        ),
    )
    vectors, areas_out, _ = run(verts_f, faces_f)
    return (vectors, areas_out)
